# Initial kernel scaffold; baseline (speedup 1.0000x reference)
import jax
import jax.numpy as jnp
from jax.experimental import pallas as pl


def kernel(x, c1_w, c1_scale, c1_shift, l1b0_c1_w, l1b0_c1_scale, l1b0_c1_shift, l1b0_c2_w, l1b0_c2_scale, l1b0_c2_shift, l1b0_c3_w, l1b0_c3_scale, l1b0_c3_shift, l1b0_ds_w, l1b0_ds_scale, l1b0_ds_shift, l1b1_c1_w, l1b1_c1_scale, l1b1_c1_shift, l1b1_c2_w, l1b1_c2_scale, l1b1_c2_shift, l1b1_c3_w, l1b1_c3_scale, l1b1_c3_shift, l1b2_c1_w, l1b2_c1_scale, l1b2_c1_shift, l1b2_c2_w, l1b2_c2_scale, l1b2_c2_shift, l1b2_c3_w, l1b2_c3_scale, l1b2_c3_shift, l2b0_c1_w, l2b0_c1_scale, l2b0_c1_shift, l2b0_c2_w, l2b0_c2_scale, l2b0_c2_shift, l2b0_c3_w, l2b0_c3_scale, l2b0_c3_shift, l2b0_ds_w, l2b0_ds_scale, l2b0_ds_shift, l2b1_c1_w, l2b1_c1_scale, l2b1_c1_shift, l2b1_c2_w, l2b1_c2_scale, l2b1_c2_shift, l2b1_c3_w, l2b1_c3_scale, l2b1_c3_shift, l2b2_c1_w, l2b2_c1_scale, l2b2_c1_shift, l2b2_c2_w, l2b2_c2_scale, l2b2_c2_shift, l2b2_c3_w, l2b2_c3_scale, l2b2_c3_shift, l2b3_c1_w, l2b3_c1_scale, l2b3_c1_shift, l2b3_c2_w, l2b3_c2_scale, l2b3_c2_shift, l2b3_c3_w, l2b3_c3_scale, l2b3_c3_shift, l3b0_c1_w, l3b0_c1_scale, l3b0_c1_shift, l3b0_c2_w, l3b0_c2_scale, l3b0_c2_shift, l3b0_c3_w, l3b0_c3_scale, l3b0_c3_shift, l3b0_ds_w, l3b0_ds_scale, l3b0_ds_shift, l3b1_c1_w, l3b1_c1_scale, l3b1_c1_shift, l3b1_c2_w, l3b1_c2_scale, l3b1_c2_shift, l3b1_c3_w, l3b1_c3_scale, l3b1_c3_shift, l3b2_c1_w, l3b2_c1_scale, l3b2_c1_shift, l3b2_c2_w, l3b2_c2_scale, l3b2_c2_shift, l3b2_c3_w, l3b2_c3_scale, l3b2_c3_shift, l3b3_c1_w, l3b3_c1_scale, l3b3_c1_shift, l3b3_c2_w, l3b3_c2_scale, l3b3_c2_shift, l3b3_c3_w, l3b3_c3_scale, l3b3_c3_shift, l3b4_c1_w, l3b4_c1_scale, l3b4_c1_shift, l3b4_c2_w, l3b4_c2_scale, l3b4_c2_shift, l3b4_c3_w, l3b4_c3_scale, l3b4_c3_shift, l3b5_c1_w, l3b5_c1_scale, l3b5_c1_shift, l3b5_c2_w, l3b5_c2_scale, l3b5_c2_shift, l3b5_c3_w, l3b5_c3_scale, l3b5_c3_shift, l4b0_c1_w, l4b0_c1_scale, l4b0_c1_shift, l4b0_c2_w, l4b0_c2_scale, l4b0_c2_shift, l4b0_c3_w, l4b0_c3_scale, l4b0_c3_shift, l4b0_ds_w, l4b0_ds_scale, l4b0_ds_shift, l4b1_c1_w, l4b1_c1_scale, l4b1_c1_shift, l4b1_c2_w, l4b1_c2_scale, l4b1_c2_shift, l4b1_c3_w, l4b1_c3_scale, l4b1_c3_shift, l4b2_c1_w, l4b2_c1_scale, l4b2_c1_shift, l4b2_c2_w, l4b2_c2_scale, l4b2_c2_shift, l4b2_c3_w, l4b2_c3_scale, l4b2_c3_shift, fc_w, fc_b):
    raise NotImplementedError("write your pallas kernel here")


# speedup vs baseline: 1.4862x; 1.4862x over previous
import functools
import jax
import jax.numpy as jnp
from jax.experimental import pallas as pl
from jax.experimental.pallas import tpu as pltpu

_VMEM_LIMIT = 32 * 1024 * 1024


def _ceil_to(x, m):
    return ((x + m - 1) // m) * m


def _mm_body(a_ref, w_ref, s_ref, t_ref, o_ref, *, act):
    y = jnp.dot(a_ref[...], w_ref[...], preferred_element_type=jnp.float32)
    y = y * s_ref[...] + t_ref[...]
    if act == "relu":
        y = jnp.maximum(y, 0.0)
    o_ref[...] = y.astype(o_ref.dtype)


def _mm_res_body(a_ref, w_ref, s_ref, t_ref, r_ref, o_ref, *, act):
    y = jnp.dot(a_ref[...], w_ref[...], preferred_element_type=jnp.float32)
    y = y * s_ref[...] + t_ref[...]
    y = y + r_ref[...].astype(jnp.float32)
    if act == "relu":
        y = jnp.maximum(y, 0.0)
    o_ref[...] = y.astype(o_ref.dtype)


def _mm(a, w, scale, shift, act="none", residual=None, out_dtype=jnp.bfloat16):
    M, K = a.shape
    N = w.shape[1]
    if M % 784 == 0:
        tm = 784
    else:
        tm = M
    tn = min(512, N)
    s2 = scale.astype(jnp.float32).reshape(1, N)
    t2 = shift.astype(jnp.float32).reshape(1, N)

    inputs = [a, w, s2, t2]
    in_specs = [
        pl.BlockSpec((tm, K), lambda i, j: (i, 0)),
        pl.BlockSpec((K, tn), lambda i, j: (0, j)),
        pl.BlockSpec((1, tn), lambda i, j: (0, j)),
        pl.BlockSpec((1, tn), lambda i, j: (0, j)),
    ]
    if residual is not None:
        body = functools.partial(_mm_res_body, act=act)
        inputs.append(residual)
        in_specs.append(pl.BlockSpec((tm, tn), lambda i, j: (i, j)))
    else:
        body = functools.partial(_mm_body, act=act)

    return pl.pallas_call(
        body,
        grid=(M // tm, N // tn),
        in_specs=in_specs,
        out_specs=pl.BlockSpec((tm, tn), lambda i, j: (i, j)),
        out_shape=jax.ShapeDtypeStruct((M, N), out_dtype),
        compiler_params=pltpu.CompilerParams(
            dimension_semantics=("parallel", "parallel"),
            vmem_limit_bytes=_VMEM_LIMIT),
    )(*inputs)


def _c3_body(x_ref, w_ref, s_ref, t_ref, o_ref, *, H, W, Wp, F, Fp):
    C = x_ref.shape[3]
    M2 = H * Wp
    p = jnp.dot(x_ref[0, 0:H, :, :].reshape(M2, C), w_ref[0],
                preferred_element_type=jnp.float32)
    p = p + jnp.dot(x_ref[0, 1:H + 1, :, :].reshape(M2, C), w_ref[1],
                    preferred_element_type=jnp.float32)
    p = p + jnp.dot(x_ref[0, 2:H + 2, :, :].reshape(M2, C), w_ref[2],
                    preferred_element_type=jnp.float32)
    p = p.reshape(H, Wp, 3 * Fp)
    acc = (p[:, 0:W, 0:Fp] + p[:, 1:W + 1, Fp:2 * Fp]
           + p[:, 2:W + 2, 2 * Fp:3 * Fp])
    y = jnp.maximum(acc * s_ref[...] + t_ref[...], 0.0)
    o_ref[0] = y[:, :, 0:F].astype(o_ref.dtype)


def _conv3_s1(x, w, scale, shift):
    Nb, H, W, C = x.shape
    F = w.shape[1]
    Wp = _ceil_to(W + 2, 16)
    Fp = max(F, 128)
    xp = jnp.pad(x, ((0, 0), (1, 1), (1, Wp - W - 1), (0, 0)))
    wt = jnp.transpose(w.reshape(3, 3, C, F), (0, 2, 1, 3))
    if Fp != F:
        wt = jnp.pad(wt, ((0, 0), (0, 0), (0, 0), (0, Fp - F)))
    ws = wt.reshape(3, C, 3 * Fp)
    sp = jnp.pad(scale.astype(jnp.float32), (0, Fp - F)).reshape(1, 1, Fp)
    tp = jnp.pad(shift.astype(jnp.float32), (0, Fp - F)).reshape(1, 1, Fp)

    return pl.pallas_call(
        functools.partial(_c3_body, H=H, W=W, Wp=Wp, F=F, Fp=Fp),
        grid=(Nb,),
        in_specs=[
            pl.BlockSpec((1, H + 2, Wp, C), lambda n: (n, 0, 0, 0)),
            pl.BlockSpec((3, C, 3 * Fp), lambda n: (0, 0, 0)),
            pl.BlockSpec((1, 1, Fp), lambda n: (0, 0, 0)),
            pl.BlockSpec((1, 1, Fp), lambda n: (0, 0, 0)),
        ],
        out_specs=pl.BlockSpec((1, H, W, F), lambda n: (n, 0, 0, 0)),
        out_shape=jax.ShapeDtypeStruct((Nb, H, W, F), jnp.bfloat16),
        compiler_params=pltpu.CompilerParams(
            dimension_semantics=("parallel",),
            vmem_limit_bytes=_VMEM_LIMIT),
    )(xp, ws, sp, tp)


def _mp_body(a00, a01, a10, a11, o_ref):
    Ho = o_ref.shape[1]
    Wo = o_ref.shape[2]

    def sl(a, r, c):
        return a[0, r:r + Ho, c:c + Wo, :]

    m = sl(a00, 0, 0)
    for a, r, c in ((a01, 0, 0), (a00, 0, 1), (a10, 0, 0), (a11, 0, 0),
                    (a10, 0, 1), (a00, 1, 0), (a01, 1, 0), (a00, 1, 1)):
        m = jnp.maximum(m, sl(a, r, c))
    o_ref[0] = m


def _maxpool_3x3_s2(x):
    Nb, H, W, C = x.shape
    Ho = (H + 2 - 3) // 2 + 1
    Wo = (W + 2 - 3) // 2 + 1
    xp = jnp.pad(x, ((0, 0), (1, 3), (1, 3), (0, 0)),
                 constant_values=float("-inf"))
    Hh = Ho + 2
    planes = [xp[:, a::2, b::2, :][:, :Hh, :Hh, :]
              for a in (0, 1) for b in (0, 1)]
    return pl.pallas_call(
        _mp_body,
        grid=(Nb,),
        in_specs=[pl.BlockSpec((1, Hh, Hh, C), lambda n: (n, 0, 0, 0))] * 4,
        out_specs=pl.BlockSpec((1, Ho, Wo, C), lambda n: (n, 0, 0, 0)),
        out_shape=jax.ShapeDtypeStruct((Nb, Ho, Wo, C), x.dtype),
        compiler_params=pltpu.CompilerParams(
            dimension_semantics=("parallel",),
            vmem_limit_bytes=_VMEM_LIMIT),
    )(*planes)


def _head_body(x_ref, w_ref, b_ref, o_ref, *, HW):
    xs = jnp.sum(x_ref[...].astype(jnp.float32), axis=1)
    pooled = (xs * (1.0 / HW)).astype(jnp.bfloat16).astype(jnp.float32)
    wv = w_ref[...].astype(jnp.float32)
    logit = jnp.sum(pooled * wv, axis=1, keepdims=True) + b_ref[...]
    o_ref[...] = 1.0 / (1.0 + jnp.exp(-logit))


def _head(x, fc_w, fc_b):
    Nb, H, W, C = x.shape
    x3 = x.reshape(Nb, H * W, C)
    wv = fc_w.reshape(1, C)
    bv = fc_b.astype(jnp.float32).reshape(1, 1)
    return pl.pallas_call(
        functools.partial(_head_body, HW=H * W),
        grid=(1,),
        in_specs=[
            pl.BlockSpec((Nb, H * W, C), lambda i: (0, 0, 0)),
            pl.BlockSpec((1, C), lambda i: (0, 0)),
            pl.BlockSpec((1, 1), lambda i: (0, 0)),
        ],
        out_specs=pl.BlockSpec((Nb, 1), lambda i: (0, 0)),
        out_shape=jax.ShapeDtypeStruct((Nb, 1), jnp.float32),
        compiler_params=pltpu.CompilerParams(
            dimension_semantics=("arbitrary",),
            vmem_limit_bytes=_VMEM_LIMIT),
    )(x3, wv, bv)


def _im2col(x, k, stride, pad):
    Nb, H, W, C = x.shape
    Ho = (H + 2 * pad - k) // stride + 1
    Wo = (W + 2 * pad - k) // stride + 1
    xp = jnp.pad(x, ((0, 0), (pad, pad), (pad, pad), (0, 0)))
    cols = [xp[:, dy:dy + stride * (Ho - 1) + 1:stride,
               dx:dx + stride * (Wo - 1) + 1:stride, :]
            for dy in range(k) for dx in range(k)]
    patches = jnp.stack(cols, axis=3)
    return patches.reshape(Nb * Ho * Wo, k * k * C), (Nb, Ho, Wo)


def _bottleneck(x, blk, stride):
    Nb, H, W, Cin = x.shape
    x2d = x.reshape(-1, Cin)
    w1, s1, t1 = blk["conv1"]
    w2, s2, t2 = blk["conv2"]
    w3, s3, t3 = blk["conv3"]
    planes = w1.shape[1]

    u = _mm(x2d, w1, s1, t1, act="relu")
    u = u.reshape(Nb, H, W, planes)
    if stride == 1:
        v = _conv3_s1(u, w2, s2, t2)
    else:
        a, (nb, ho, wo) = _im2col(u, 3, stride, 1)
        v = _mm(a, w2, s2, t2, act="relu").reshape(nb, ho, wo, planes)
    Ho, Wo = v.shape[1], v.shape[2]

    if "ds" in blk:
        wd, sd, td = blk["ds"]
        xs = x[:, ::stride, ::stride, :] if stride > 1 else x
        ident = _mm(xs.reshape(-1, Cin), wd, sd, td, act="none")
    else:
        ident = x2d
    out = _mm(v.reshape(-1, planes), w3, s3, t3, act="relu", residual=ident)
    return out.reshape(Nb, Ho, Wo, 4 * planes)


def kernel(x, c1_w, c1_scale, c1_shift, l1b0_c1_w, l1b0_c1_scale, l1b0_c1_shift, l1b0_c2_w, l1b0_c2_scale, l1b0_c2_shift, l1b0_c3_w, l1b0_c3_scale, l1b0_c3_shift, l1b0_ds_w, l1b0_ds_scale, l1b0_ds_shift, l1b1_c1_w, l1b1_c1_scale, l1b1_c1_shift, l1b1_c2_w, l1b1_c2_scale, l1b1_c2_shift, l1b1_c3_w, l1b1_c3_scale, l1b1_c3_shift, l1b2_c1_w, l1b2_c1_scale, l1b2_c1_shift, l1b2_c2_w, l1b2_c2_scale, l1b2_c2_shift, l1b2_c3_w, l1b2_c3_scale, l1b2_c3_shift, l2b0_c1_w, l2b0_c1_scale, l2b0_c1_shift, l2b0_c2_w, l2b0_c2_scale, l2b0_c2_shift, l2b0_c3_w, l2b0_c3_scale, l2b0_c3_shift, l2b0_ds_w, l2b0_ds_scale, l2b0_ds_shift, l2b1_c1_w, l2b1_c1_scale, l2b1_c1_shift, l2b1_c2_w, l2b1_c2_scale, l2b1_c2_shift, l2b1_c3_w, l2b1_c3_scale, l2b1_c3_shift, l2b2_c1_w, l2b2_c1_scale, l2b2_c1_shift, l2b2_c2_w, l2b2_c2_scale, l2b2_c2_shift, l2b2_c3_w, l2b2_c3_scale, l2b2_c3_shift, l2b3_c1_w, l2b3_c1_scale, l2b3_c1_shift, l2b3_c2_w, l2b3_c2_scale, l2b3_c2_shift, l2b3_c3_w, l2b3_c3_scale, l2b3_c3_shift, l3b0_c1_w, l3b0_c1_scale, l3b0_c1_shift, l3b0_c2_w, l3b0_c2_scale, l3b0_c2_shift, l3b0_c3_w, l3b0_c3_scale, l3b0_c3_shift, l3b0_ds_w, l3b0_ds_scale, l3b0_ds_shift, l3b1_c1_w, l3b1_c1_scale, l3b1_c1_shift, l3b1_c2_w, l3b1_c2_scale, l3b1_c2_shift, l3b1_c3_w, l3b1_c3_scale, l3b1_c3_shift, l3b2_c1_w, l3b2_c1_scale, l3b2_c1_shift, l3b2_c2_w, l3b2_c2_scale, l3b2_c2_shift, l3b2_c3_w, l3b2_c3_scale, l3b2_c3_shift, l3b3_c1_w, l3b3_c1_scale, l3b3_c1_shift, l3b3_c2_w, l3b3_c2_scale, l3b3_c2_shift, l3b3_c3_w, l3b3_c3_scale, l3b3_c3_shift, l3b4_c1_w, l3b4_c1_scale, l3b4_c1_shift, l3b4_c2_w, l3b4_c2_scale, l3b4_c2_shift, l3b4_c3_w, l3b4_c3_scale, l3b4_c3_shift, l3b5_c1_w, l3b5_c1_scale, l3b5_c1_shift, l3b5_c2_w, l3b5_c2_scale, l3b5_c2_shift, l3b5_c3_w, l3b5_c3_scale, l3b5_c3_shift, l4b0_c1_w, l4b0_c1_scale, l4b0_c1_shift, l4b0_c2_w, l4b0_c2_scale, l4b0_c2_shift, l4b0_c3_w, l4b0_c3_scale, l4b0_c3_shift, l4b0_ds_w, l4b0_ds_scale, l4b0_ds_shift, l4b1_c1_w, l4b1_c1_scale, l4b1_c1_shift, l4b1_c2_w, l4b1_c2_scale, l4b1_c2_shift, l4b1_c3_w, l4b1_c3_scale, l4b1_c3_shift, l4b2_c1_w, l4b2_c1_scale, l4b2_c1_shift, l4b2_c2_w, l4b2_c2_scale, l4b2_c2_shift, l4b2_c3_w, l4b2_c3_scale, l4b2_c3_shift, fc_w, fc_b):
    lv = locals()
    layer_blocks = [("l1", 3), ("l2", 4), ("l3", 6), ("l4", 3)]
    layers = []
    for lname, nblk in layer_blocks:
        blocks = []
        for b in range(nblk):
            pre = "%sb%d" % (lname, b)
            blk = {}
            for cn in ("c1", "c2", "c3"):
                blk[cn.replace("c", "conv")] = (
                    lv["%s_%s_w" % (pre, cn)],
                    lv["%s_%s_scale" % (pre, cn)],
                    lv["%s_%s_shift" % (pre, cn)])
            if ("%s_ds_w" % pre) in lv:
                blk["ds"] = (lv["%s_ds_w" % pre],
                             lv["%s_ds_scale" % pre],
                             lv["%s_ds_shift" % pre])
            blocks.append(blk)
        layers.append(blocks)

    h = jnp.transpose(x, (0, 2, 3, 1)).astype(jnp.bfloat16)
    a, (nb, ho, wo) = _im2col(h, 7, 2, 3)
    h = _mm(a, c1_w, c1_scale, c1_shift, act="relu").reshape(nb, ho, wo, 64)
    h = _maxpool_3x3_s2(h)
    strides = [1, 2, 2, 2]
    for li, blocks in enumerate(layers):
        for b, blk in enumerate(blocks):
            h = _bottleneck(h, blk, strides[li] if b == 0 else 1)
    return _head(h, fc_w, fc_b)


# speedup vs baseline: 2.2502x; 1.5141x over previous
import functools
import jax
import jax.numpy as jnp
from jax.experimental import pallas as pl
from jax.experimental.pallas import tpu as pltpu

_VMEM_LIMIT = 32 * 1024 * 1024


def _ceil_to(x, m):
    return ((x + m - 1) // m) * m


def _mm_body(a_ref, w_ref, s_ref, t_ref, o_ref, *, act):
    y = jnp.dot(a_ref[...], w_ref[...], preferred_element_type=jnp.float32)
    y = y * s_ref[...] + t_ref[...]
    if act == "relu":
        y = jnp.maximum(y, 0.0)
    o_ref[...] = y.astype(o_ref.dtype)


def _mm_res_body(a_ref, w_ref, s_ref, t_ref, r_ref, o_ref, *, act):
    y = jnp.dot(a_ref[...], w_ref[...], preferred_element_type=jnp.float32)
    y = y * s_ref[...] + t_ref[...]
    y = y + r_ref[...].astype(jnp.float32)
    if act == "relu":
        y = jnp.maximum(y, 0.0)
    o_ref[...] = y.astype(o_ref.dtype)


def _mm(a, w, scale, shift, act="none", residual=None, out_dtype=jnp.bfloat16):
    M, K = a.shape
    N = w.shape[1]
    if M % 784 == 0:
        tm = 784
    else:
        tm = M
    tn = min(512, N)
    s2 = scale.astype(jnp.float32).reshape(1, N)
    t2 = shift.astype(jnp.float32).reshape(1, N)

    inputs = [a, w, s2, t2]
    in_specs = [
        pl.BlockSpec((tm, K), lambda i, j: (i, 0)),
        pl.BlockSpec((K, tn), lambda i, j: (0, j)),
        pl.BlockSpec((1, tn), lambda i, j: (0, j)),
        pl.BlockSpec((1, tn), lambda i, j: (0, j)),
    ]
    if residual is not None:
        body = functools.partial(_mm_res_body, act=act)
        inputs.append(residual)
        in_specs.append(pl.BlockSpec((tm, tn), lambda i, j: (i, j)))
    else:
        body = functools.partial(_mm_body, act=act)

    return pl.pallas_call(
        body,
        grid=(M // tm, N // tn),
        in_specs=in_specs,
        out_specs=pl.BlockSpec((tm, tn), lambda i, j: (i, j)),
        out_shape=jax.ShapeDtypeStruct((M, N), out_dtype),
        compiler_params=pltpu.CompilerParams(
            dimension_semantics=("parallel", "parallel"),
            vmem_limit_bytes=_VMEM_LIMIT),
    )(*inputs)


def _c3_body(x_ref, w_ref, s_ref, t_ref, o_ref, *, H, W, Wp, F, Fp):
    C = x_ref.shape[3]
    M2 = H * Wp
    p = jnp.dot(x_ref[0, 0:H, :, :].reshape(M2, C), w_ref[0],
                preferred_element_type=jnp.float32)
    p = p + jnp.dot(x_ref[0, 1:H + 1, :, :].reshape(M2, C), w_ref[1],
                    preferred_element_type=jnp.float32)
    p = p + jnp.dot(x_ref[0, 2:H + 2, :, :].reshape(M2, C), w_ref[2],
                    preferred_element_type=jnp.float32)
    p = p.reshape(H, Wp, 3 * Fp)
    acc = (p[:, 0:W, 0:Fp] + p[:, 1:W + 1, Fp:2 * Fp]
           + p[:, 2:W + 2, 2 * Fp:3 * Fp])
    y = jnp.maximum(acc * s_ref[...] + t_ref[...], 0.0)
    o_ref[0] = y[:, :, 0:F].astype(o_ref.dtype)


def _conv3_s1(x, w, scale, shift):
    Nb, H, W, C = x.shape
    F = w.shape[1]
    Wp = _ceil_to(W + 2, 16)
    Fp = max(F, 128)
    xp = jnp.pad(x, ((0, 0), (1, 1), (1, Wp - W - 1), (0, 0)))
    wt = jnp.transpose(w.reshape(3, 3, C, F), (0, 2, 1, 3))
    if Fp != F:
        wt = jnp.pad(wt, ((0, 0), (0, 0), (0, 0), (0, Fp - F)))
    ws = wt.reshape(3, C, 3 * Fp)
    sp = jnp.pad(scale.astype(jnp.float32), (0, Fp - F)).reshape(1, 1, Fp)
    tp = jnp.pad(shift.astype(jnp.float32), (0, Fp - F)).reshape(1, 1, Fp)

    return pl.pallas_call(
        functools.partial(_c3_body, H=H, W=W, Wp=Wp, F=F, Fp=Fp),
        grid=(Nb,),
        in_specs=[
            pl.BlockSpec((1, H + 2, Wp, C), lambda n: (n, 0, 0, 0)),
            pl.BlockSpec((3, C, 3 * Fp), lambda n: (0, 0, 0)),
            pl.BlockSpec((1, 1, Fp), lambda n: (0, 0, 0)),
            pl.BlockSpec((1, 1, Fp), lambda n: (0, 0, 0)),
        ],
        out_specs=pl.BlockSpec((1, H, W, F), lambda n: (n, 0, 0, 0)),
        out_shape=jax.ShapeDtypeStruct((Nb, H, W, F), jnp.bfloat16),
        compiler_params=pltpu.CompilerParams(
            dimension_semantics=("parallel",),
            vmem_limit_bytes=_VMEM_LIMIT),
    )(xp, ws, sp, tp)


def _stem_body(x_ref, w_ref, s_ref, t_ref, o_ref, *, Wo, Wp):
    th = x_ref.shape[1]
    A = x_ref[0].reshape(th * Wp, 48)
    p = jnp.dot(A, w_ref[...], preferred_element_type=jnp.float32)
    p = p.reshape(th, Wp, 256)
    acc = (p[:, 0:Wo, 0:64] + p[:, 1:Wo + 1, 64:128]
           + p[:, 2:Wo + 2, 128:192] + p[:, 3:Wo + 3, 192:256])
    y = jnp.maximum(acc * s_ref[...] + t_ref[...], 0.0)
    o_ref[0] = y.astype(o_ref.dtype)


def _stem_conv(xh, w, scale, shift):
    Nb, H, W, _ = xh.shape
    Ho, Wo = H // 2, W // 2
    Wp = _ceil_to(Wo + 3, 16)
    xp = jnp.pad(xh, ((0, 0), (3, 3), (3, 3), (0, 0)))
    planes = [jnp.pad(xp[:, a::2, b::2, :][:, :Ho + 3, :Wo + 3, :],
                      ((0, 0), (0, 0), (0, Wp - Wo - 3), (0, 0)))
              for a in (0, 1) for b in (0, 1)]
    xs = jnp.concatenate(planes, axis=-1)
    xss = jnp.concatenate([xs[:, j:j + Ho] for j in range(4)],
                          axis=-1)
    w6 = jnp.zeros((8, 8, 3, 64), jnp.bfloat16).at[:7, :7].set(
        w.reshape(7, 7, 3, 64))
    wc = jnp.transpose(w6.reshape(4, 2, 4, 2, 3, 64),
                       (0, 1, 3, 4, 2, 5)).reshape(48, 256)
    sp = scale.astype(jnp.float32).reshape(1, 1, 64)
    tp = shift.astype(jnp.float32).reshape(1, 1, 64)
    th = 28 if Ho % 28 == 0 else Ho
    return pl.pallas_call(
        functools.partial(_stem_body, Wo=Wo, Wp=Wp),
        grid=(Nb, Ho // th),
        in_specs=[
            pl.BlockSpec((1, th, Wp, 48), lambda n, i: (n, i, 0, 0)),
            pl.BlockSpec((48, 256), lambda n, i: (0, 0)),
            pl.BlockSpec((1, 1, 64), lambda n, i: (0, 0, 0)),
            pl.BlockSpec((1, 1, 64), lambda n, i: (0, 0, 0)),
        ],
        out_specs=pl.BlockSpec((1, th, Wo, 64), lambda n, i: (n, i, 0, 0)),
        out_shape=jax.ShapeDtypeStruct((Nb, Ho, Wo, 64), jnp.bfloat16),
        compiler_params=pltpu.CompilerParams(
            dimension_semantics=("parallel", "parallel"),
            vmem_limit_bytes=_VMEM_LIMIT),
    )(xss, wc, sp, tp)


def _mp_body(a00, a01, a10, a11, o_ref):
    Ho = o_ref.shape[1]
    Wo = o_ref.shape[2]

    def sl(a, r, c):
        return a[0, r:r + Ho, c:c + Wo, :]

    m = sl(a00, 0, 0)
    for a, r, c in ((a01, 0, 0), (a00, 0, 1), (a10, 0, 0), (a11, 0, 0),
                    (a10, 0, 1), (a00, 1, 0), (a01, 1, 0), (a00, 1, 1)):
        m = jnp.maximum(m, sl(a, r, c))
    o_ref[0] = m


def _maxpool_3x3_s2(x):
    Nb, H, W, C = x.shape
    Ho = (H + 2 - 3) // 2 + 1
    Wo = (W + 2 - 3) // 2 + 1
    xp = jnp.pad(x, ((0, 0), (1, 3), (1, 3), (0, 0)),
                 constant_values=float("-inf"))
    Hh = Ho + 2
    planes = [xp[:, a::2, b::2, :][:, :Hh, :Hh, :]
              for a in (0, 1) for b in (0, 1)]
    return pl.pallas_call(
        _mp_body,
        grid=(Nb,),
        in_specs=[pl.BlockSpec((1, Hh, Hh, C), lambda n: (n, 0, 0, 0))] * 4,
        out_specs=pl.BlockSpec((1, Ho, Wo, C), lambda n: (n, 0, 0, 0)),
        out_shape=jax.ShapeDtypeStruct((Nb, Ho, Wo, C), x.dtype),
        compiler_params=pltpu.CompilerParams(
            dimension_semantics=("parallel",),
            vmem_limit_bytes=_VMEM_LIMIT),
    )(*planes)


def _head_body(x_ref, w_ref, b_ref, o_ref, *, HW):
    xs = jnp.sum(x_ref[...].astype(jnp.float32), axis=1)
    pooled = (xs * (1.0 / HW)).astype(jnp.bfloat16).astype(jnp.float32)
    wv = w_ref[...].astype(jnp.float32)
    logit = jnp.sum(pooled * wv, axis=1, keepdims=True) + b_ref[...]
    o_ref[...] = 1.0 / (1.0 + jnp.exp(-logit))


def _head(x, fc_w, fc_b):
    Nb, H, W, C = x.shape
    x3 = x.reshape(Nb, H * W, C)
    wv = fc_w.reshape(1, C)
    bv = fc_b.astype(jnp.float32).reshape(1, 1)
    return pl.pallas_call(
        functools.partial(_head_body, HW=H * W),
        grid=(1,),
        in_specs=[
            pl.BlockSpec((Nb, H * W, C), lambda i: (0, 0, 0)),
            pl.BlockSpec((1, C), lambda i: (0, 0)),
            pl.BlockSpec((1, 1), lambda i: (0, 0)),
        ],
        out_specs=pl.BlockSpec((Nb, 1), lambda i: (0, 0)),
        out_shape=jax.ShapeDtypeStruct((Nb, 1), jnp.float32),
        compiler_params=pltpu.CompilerParams(
            dimension_semantics=("arbitrary",),
            vmem_limit_bytes=_VMEM_LIMIT),
    )(x3, wv, bv)


def _im2col(x, k, stride, pad):
    Nb, H, W, C = x.shape
    Ho = (H + 2 * pad - k) // stride + 1
    Wo = (W + 2 * pad - k) // stride + 1
    xp = jnp.pad(x, ((0, 0), (pad, pad), (pad, pad), (0, 0)))
    cols = [xp[:, dy:dy + stride * (Ho - 1) + 1:stride,
               dx:dx + stride * (Wo - 1) + 1:stride, :]
            for dy in range(k) for dx in range(k)]
    patches = jnp.stack(cols, axis=3)
    return patches.reshape(Nb * Ho * Wo, k * k * C), (Nb, Ho, Wo)


def _bottleneck(x, blk, stride):
    Nb, H, W, Cin = x.shape
    x2d = x.reshape(-1, Cin)
    w1, s1, t1 = blk["conv1"]
    w2, s2, t2 = blk["conv2"]
    w3, s3, t3 = blk["conv3"]
    planes = w1.shape[1]

    u = _mm(x2d, w1, s1, t1, act="relu")
    u = u.reshape(Nb, H, W, planes)
    if stride == 1:
        v = _conv3_s1(u, w2, s2, t2)
    else:
        a, (nb, ho, wo) = _im2col(u, 3, stride, 1)
        v = _mm(a, w2, s2, t2, act="relu").reshape(nb, ho, wo, planes)
    Ho, Wo = v.shape[1], v.shape[2]

    if "ds" in blk:
        wd, sd, td = blk["ds"]
        xs = x[:, ::stride, ::stride, :] if stride > 1 else x
        ident = _mm(xs.reshape(-1, Cin), wd, sd, td, act="none")
    else:
        ident = x2d
    out = _mm(v.reshape(-1, planes), w3, s3, t3, act="relu", residual=ident)
    return out.reshape(Nb, Ho, Wo, 4 * planes)


def kernel(x, c1_w, c1_scale, c1_shift, l1b0_c1_w, l1b0_c1_scale, l1b0_c1_shift, l1b0_c2_w, l1b0_c2_scale, l1b0_c2_shift, l1b0_c3_w, l1b0_c3_scale, l1b0_c3_shift, l1b0_ds_w, l1b0_ds_scale, l1b0_ds_shift, l1b1_c1_w, l1b1_c1_scale, l1b1_c1_shift, l1b1_c2_w, l1b1_c2_scale, l1b1_c2_shift, l1b1_c3_w, l1b1_c3_scale, l1b1_c3_shift, l1b2_c1_w, l1b2_c1_scale, l1b2_c1_shift, l1b2_c2_w, l1b2_c2_scale, l1b2_c2_shift, l1b2_c3_w, l1b2_c3_scale, l1b2_c3_shift, l2b0_c1_w, l2b0_c1_scale, l2b0_c1_shift, l2b0_c2_w, l2b0_c2_scale, l2b0_c2_shift, l2b0_c3_w, l2b0_c3_scale, l2b0_c3_shift, l2b0_ds_w, l2b0_ds_scale, l2b0_ds_shift, l2b1_c1_w, l2b1_c1_scale, l2b1_c1_shift, l2b1_c2_w, l2b1_c2_scale, l2b1_c2_shift, l2b1_c3_w, l2b1_c3_scale, l2b1_c3_shift, l2b2_c1_w, l2b2_c1_scale, l2b2_c1_shift, l2b2_c2_w, l2b2_c2_scale, l2b2_c2_shift, l2b2_c3_w, l2b2_c3_scale, l2b2_c3_shift, l2b3_c1_w, l2b3_c1_scale, l2b3_c1_shift, l2b3_c2_w, l2b3_c2_scale, l2b3_c2_shift, l2b3_c3_w, l2b3_c3_scale, l2b3_c3_shift, l3b0_c1_w, l3b0_c1_scale, l3b0_c1_shift, l3b0_c2_w, l3b0_c2_scale, l3b0_c2_shift, l3b0_c3_w, l3b0_c3_scale, l3b0_c3_shift, l3b0_ds_w, l3b0_ds_scale, l3b0_ds_shift, l3b1_c1_w, l3b1_c1_scale, l3b1_c1_shift, l3b1_c2_w, l3b1_c2_scale, l3b1_c2_shift, l3b1_c3_w, l3b1_c3_scale, l3b1_c3_shift, l3b2_c1_w, l3b2_c1_scale, l3b2_c1_shift, l3b2_c2_w, l3b2_c2_scale, l3b2_c2_shift, l3b2_c3_w, l3b2_c3_scale, l3b2_c3_shift, l3b3_c1_w, l3b3_c1_scale, l3b3_c1_shift, l3b3_c2_w, l3b3_c2_scale, l3b3_c2_shift, l3b3_c3_w, l3b3_c3_scale, l3b3_c3_shift, l3b4_c1_w, l3b4_c1_scale, l3b4_c1_shift, l3b4_c2_w, l3b4_c2_scale, l3b4_c2_shift, l3b4_c3_w, l3b4_c3_scale, l3b4_c3_shift, l3b5_c1_w, l3b5_c1_scale, l3b5_c1_shift, l3b5_c2_w, l3b5_c2_scale, l3b5_c2_shift, l3b5_c3_w, l3b5_c3_scale, l3b5_c3_shift, l4b0_c1_w, l4b0_c1_scale, l4b0_c1_shift, l4b0_c2_w, l4b0_c2_scale, l4b0_c2_shift, l4b0_c3_w, l4b0_c3_scale, l4b0_c3_shift, l4b0_ds_w, l4b0_ds_scale, l4b0_ds_shift, l4b1_c1_w, l4b1_c1_scale, l4b1_c1_shift, l4b1_c2_w, l4b1_c2_scale, l4b1_c2_shift, l4b1_c3_w, l4b1_c3_scale, l4b1_c3_shift, l4b2_c1_w, l4b2_c1_scale, l4b2_c1_shift, l4b2_c2_w, l4b2_c2_scale, l4b2_c2_shift, l4b2_c3_w, l4b2_c3_scale, l4b2_c3_shift, fc_w, fc_b):
    lv = locals()
    layer_blocks = [("l1", 3), ("l2", 4), ("l3", 6), ("l4", 3)]
    layers = []
    for lname, nblk in layer_blocks:
        blocks = []
        for b in range(nblk):
            pre = "%sb%d" % (lname, b)
            blk = {}
            for cn in ("c1", "c2", "c3"):
                blk[cn.replace("c", "conv")] = (
                    lv["%s_%s_w" % (pre, cn)],
                    lv["%s_%s_scale" % (pre, cn)],
                    lv["%s_%s_shift" % (pre, cn)])
            if ("%s_ds_w" % pre) in lv:
                blk["ds"] = (lv["%s_ds_w" % pre],
                             lv["%s_ds_scale" % pre],
                             lv["%s_ds_shift" % pre])
            blocks.append(blk)
        layers.append(blocks)

    h = jnp.transpose(x, (0, 2, 3, 1)).astype(jnp.bfloat16)
    h = _stem_conv(h, c1_w, c1_scale, c1_shift)
    h = _maxpool_3x3_s2(h)
    strides = [1, 2, 2, 2]
    for li, blocks in enumerate(layers):
        for b, blk in enumerate(blocks):
            h = _bottleneck(h, blk, strides[li] if b == 0 else 1)
    return _head(h, fc_w, fc_b)


# speedup vs baseline: 3.3332x; 1.4812x over previous
import functools
import jax
import jax.numpy as jnp
from jax.experimental import pallas as pl
from jax.experimental.pallas import tpu as pltpu

_VMEM_LIMIT = 32 * 1024 * 1024


def _ceil_to(x, m):
    return ((x + m - 1) // m) * m


def _mm_body(a_ref, w_ref, s_ref, t_ref, o_ref, *, act):
    y = jnp.dot(a_ref[...], w_ref[...], preferred_element_type=jnp.float32)
    y = y * s_ref[...] + t_ref[...]
    if act == "relu":
        y = jnp.maximum(y, 0.0)
    o_ref[...] = y.astype(o_ref.dtype)


def _mm_res_body(a_ref, w_ref, s_ref, t_ref, r_ref, o_ref, *, act):
    y = jnp.dot(a_ref[...], w_ref[...], preferred_element_type=jnp.float32)
    y = y * s_ref[...] + t_ref[...]
    y = y + r_ref[...].astype(jnp.float32)
    if act == "relu":
        y = jnp.maximum(y, 0.0)
    o_ref[...] = y.astype(o_ref.dtype)


def _mm(a, w, scale, shift, act="none", residual=None, out_dtype=jnp.bfloat16):
    M, K = a.shape
    N = w.shape[1]
    if M % 784 == 0:
        tm = 784
    else:
        tm = M
    tn = min(512, N)
    s2 = scale.astype(jnp.float32).reshape(1, N)
    t2 = shift.astype(jnp.float32).reshape(1, N)

    inputs = [a, w, s2, t2]
    in_specs = [
        pl.BlockSpec((tm, K), lambda i, j: (i, 0)),
        pl.BlockSpec((K, tn), lambda i, j: (0, j)),
        pl.BlockSpec((1, tn), lambda i, j: (0, j)),
        pl.BlockSpec((1, tn), lambda i, j: (0, j)),
    ]
    if residual is not None:
        body = functools.partial(_mm_res_body, act=act)
        inputs.append(residual)
        in_specs.append(pl.BlockSpec((tm, tn), lambda i, j: (i, j)))
    else:
        body = functools.partial(_mm_body, act=act)

    return pl.pallas_call(
        body,
        grid=(M // tm, N // tn),
        in_specs=in_specs,
        out_specs=pl.BlockSpec((tm, tn), lambda i, j: (i, j)),
        out_shape=jax.ShapeDtypeStruct((M, N), out_dtype),
        compiler_params=pltpu.CompilerParams(
            dimension_semantics=("parallel", "parallel"),
            vmem_limit_bytes=_VMEM_LIMIT),
    )(*inputs)


def _c3_body(x_ref, w_ref, s_ref, t_ref, o_ref, *, H, W, Wp, F, Fp):
    C = x_ref.shape[3]
    M2 = H * Wp
    p = jnp.dot(x_ref[0, 0:H, :, :].reshape(M2, C), w_ref[0],
                preferred_element_type=jnp.float32)
    p = p + jnp.dot(x_ref[0, 1:H + 1, :, :].reshape(M2, C), w_ref[1],
                    preferred_element_type=jnp.float32)
    p = p + jnp.dot(x_ref[0, 2:H + 2, :, :].reshape(M2, C), w_ref[2],
                    preferred_element_type=jnp.float32)
    p = p.reshape(H, Wp, 3 * Fp)
    acc = (p[:, 0:W, 0:Fp] + p[:, 1:W + 1, Fp:2 * Fp]
           + p[:, 2:W + 2, 2 * Fp:3 * Fp])
    y = jnp.maximum(acc * s_ref[...] + t_ref[...], 0.0)
    o_ref[0] = y[:, :, 0:F].astype(o_ref.dtype)


def _conv3_s1(x, w, scale, shift):
    Nb, H, W, C = x.shape
    F = w.shape[1]
    Wp = _ceil_to(W + 2, 16)
    Fp = max(F, 128)
    xp = jnp.pad(x, ((0, 0), (1, 1), (1, Wp - W - 1), (0, 0)))
    wt = jnp.transpose(w.reshape(3, 3, C, F), (0, 2, 1, 3))
    if Fp != F:
        wt = jnp.pad(wt, ((0, 0), (0, 0), (0, 0), (0, Fp - F)))
    ws = wt.reshape(3, C, 3 * Fp)
    sp = jnp.pad(scale.astype(jnp.float32), (0, Fp - F)).reshape(1, 1, Fp)
    tp = jnp.pad(shift.astype(jnp.float32), (0, Fp - F)).reshape(1, 1, Fp)

    return pl.pallas_call(
        functools.partial(_c3_body, H=H, W=W, Wp=Wp, F=F, Fp=Fp),
        grid=(Nb,),
        in_specs=[
            pl.BlockSpec((1, H + 2, Wp, C), lambda n: (n, 0, 0, 0)),
            pl.BlockSpec((3, C, 3 * Fp), lambda n: (0, 0, 0)),
            pl.BlockSpec((1, 1, Fp), lambda n: (0, 0, 0)),
            pl.BlockSpec((1, 1, Fp), lambda n: (0, 0, 0)),
        ],
        out_specs=pl.BlockSpec((1, H, W, F), lambda n: (n, 0, 0, 0)),
        out_shape=jax.ShapeDtypeStruct((Nb, H, W, F), jnp.bfloat16),
        compiler_params=pltpu.CompilerParams(
            dimension_semantics=("parallel",),
            vmem_limit_bytes=_VMEM_LIMIT),
    )(xp, ws, sp, tp)


def _stem_body(x_ref, w_ref, s_ref, t_ref, o_ref, *, Wo, Wp):
    th = x_ref.shape[1]
    A = x_ref[0].reshape(th * Wp, 48)
    p = jnp.dot(A, w_ref[...], preferred_element_type=jnp.float32)
    p = p.reshape(th, Wp, 256)
    acc = (p[:, 0:Wo, 0:64] + p[:, 1:Wo + 1, 64:128]
           + p[:, 2:Wo + 2, 128:192] + p[:, 3:Wo + 3, 192:256])
    y = jnp.maximum(acc * s_ref[...] + t_ref[...], 0.0)
    o_ref[0] = y.astype(o_ref.dtype)


def _stem_conv(xh, w, scale, shift):
    Nb, H, W, _ = xh.shape
    Ho, Wo = H // 2, W // 2
    Wp = _ceil_to(Wo + 3, 16)
    xp = jnp.pad(xh, ((0, 0), (3, 3), (3, 3), (0, 0)))
    planes = [jnp.pad(xp[:, a::2, b::2, :][:, :Ho + 3, :Wo + 3, :],
                      ((0, 0), (0, 0), (0, Wp - Wo - 3), (0, 0)))
              for a in (0, 1) for b in (0, 1)]
    xs = jnp.concatenate(planes, axis=-1)
    xss = jnp.concatenate([xs[:, j:j + Ho] for j in range(4)],
                          axis=-1)
    w6 = jnp.zeros((8, 8, 3, 64), jnp.bfloat16).at[:7, :7].set(
        w.reshape(7, 7, 3, 64))
    wc = jnp.transpose(w6.reshape(4, 2, 4, 2, 3, 64),
                       (0, 1, 3, 4, 2, 5)).reshape(48, 256)
    sp = scale.astype(jnp.float32).reshape(1, 1, 64)
    tp = shift.astype(jnp.float32).reshape(1, 1, 64)
    th = 28 if Ho % 28 == 0 else Ho
    return pl.pallas_call(
        functools.partial(_stem_body, Wo=Wo, Wp=Wp),
        grid=(Nb, Ho // th),
        in_specs=[
            pl.BlockSpec((1, th, Wp, 48), lambda n, i: (n, i, 0, 0)),
            pl.BlockSpec((48, 256), lambda n, i: (0, 0)),
            pl.BlockSpec((1, 1, 64), lambda n, i: (0, 0, 0)),
            pl.BlockSpec((1, 1, 64), lambda n, i: (0, 0, 0)),
        ],
        out_specs=pl.BlockSpec((1, th, Wo, 64), lambda n, i: (n, i, 0, 0)),
        out_shape=jax.ShapeDtypeStruct((Nb, Ho, Wo, 64), jnp.bfloat16),
        compiler_params=pltpu.CompilerParams(
            dimension_semantics=("parallel", "parallel"),
            vmem_limit_bytes=_VMEM_LIMIT),
    )(xss, wc, sp, tp)


def _mp_body(x_ref, o_ref):
    H = x_ref.shape[1]
    Wh = x_ref.shape[2]
    C = o_ref.shape[3]
    ninf = jnp.full((), float("-inf"), x_ref.dtype)
    xv = x_ref[0]
    a = xv[:, :, 0:C]
    b = xv[:, :, C:2 * C]
    bm = jnp.concatenate(
        [jnp.full((H, 1, C), ninf, xv.dtype), b[:, :Wh - 1, :]], axis=1)
    mw = jnp.maximum(jnp.maximum(a, b), bm)
    rm = jnp.concatenate([jnp.full((1, Wh, C), ninf, xv.dtype),
                          mw[:H - 1]], axis=0)
    rp = jnp.concatenate([mw[1:], jnp.full((1, Wh, C), ninf, xv.dtype)],
                         axis=0)
    m3 = jnp.maximum(jnp.maximum(mw, rm), rp)
    o_ref[0] = m3.reshape(H // 2, 2, Wh, C)[:, 0]


def _maxpool_3x3_s2(x):
    Nb, H, W, C = x.shape
    xv = x.reshape(Nb, H, W // 2, 2 * C)
    return pl.pallas_call(
        _mp_body,
        grid=(Nb,),
        in_specs=[pl.BlockSpec((1, H, W // 2, 2 * C), lambda n: (n, 0, 0, 0))],
        out_specs=pl.BlockSpec((1, H // 2, W // 2, C), lambda n: (n, 0, 0, 0)),
        out_shape=jax.ShapeDtypeStruct((Nb, H // 2, W // 2, C), x.dtype),
        compiler_params=pltpu.CompilerParams(
            dimension_semantics=("parallel",),
            vmem_limit_bytes=_VMEM_LIMIT),
    )(xv)


def _head_body(x_ref, w_ref, b_ref, o_ref, *, HW):
    xs = jnp.sum(x_ref[...].astype(jnp.float32), axis=1)
    pooled = (xs * (1.0 / HW)).astype(jnp.bfloat16).astype(jnp.float32)
    wv = w_ref[...].astype(jnp.float32)
    logit = jnp.sum(pooled * wv, axis=1, keepdims=True) + b_ref[...]
    o_ref[...] = 1.0 / (1.0 + jnp.exp(-logit))


def _head(x, fc_w, fc_b):
    Nb, H, W, C = x.shape
    x3 = x.reshape(Nb, H * W, C)
    wv = fc_w.reshape(1, C)
    bv = fc_b.astype(jnp.float32).reshape(1, 1)
    return pl.pallas_call(
        functools.partial(_head_body, HW=H * W),
        grid=(1,),
        in_specs=[
            pl.BlockSpec((Nb, H * W, C), lambda i: (0, 0, 0)),
            pl.BlockSpec((1, C), lambda i: (0, 0)),
            pl.BlockSpec((1, 1), lambda i: (0, 0)),
        ],
        out_specs=pl.BlockSpec((Nb, 1), lambda i: (0, 0)),
        out_shape=jax.ShapeDtypeStruct((Nb, 1), jnp.float32),
        compiler_params=pltpu.CompilerParams(
            dimension_semantics=("arbitrary",),
            vmem_limit_bytes=_VMEM_LIMIT),
    )(x3, wv, bv)


def _im2col(x, k, stride, pad):
    Nb, H, W, C = x.shape
    Ho = (H + 2 * pad - k) // stride + 1
    Wo = (W + 2 * pad - k) // stride + 1
    xp = jnp.pad(x, ((0, 0), (pad, pad), (pad, pad), (0, 0)))
    cols = [xp[:, dy:dy + stride * (Ho - 1) + 1:stride,
               dx:dx + stride * (Wo - 1) + 1:stride, :]
            for dy in range(k) for dx in range(k)]
    patches = jnp.stack(cols, axis=3)
    return patches.reshape(Nb * Ho * Wo, k * k * C), (Nb, Ho, Wo)


def _bottleneck(x, blk, stride):
    Nb, H, W, Cin = x.shape
    x2d = x.reshape(-1, Cin)
    w1, s1, t1 = blk["conv1"]
    w2, s2, t2 = blk["conv2"]
    w3, s3, t3 = blk["conv3"]
    planes = w1.shape[1]

    u = _mm(x2d, w1, s1, t1, act="relu")
    u = u.reshape(Nb, H, W, planes)
    if stride == 1:
        v = _conv3_s1(u, w2, s2, t2)
    else:
        a, (nb, ho, wo) = _im2col(u, 3, stride, 1)
        v = _mm(a, w2, s2, t2, act="relu").reshape(nb, ho, wo, planes)
    Ho, Wo = v.shape[1], v.shape[2]

    if "ds" in blk:
        wd, sd, td = blk["ds"]
        xs = x[:, ::stride, ::stride, :] if stride > 1 else x
        ident = _mm(xs.reshape(-1, Cin), wd, sd, td, act="none")
    else:
        ident = x2d
    out = _mm(v.reshape(-1, planes), w3, s3, t3, act="relu", residual=ident)
    return out.reshape(Nb, Ho, Wo, 4 * planes)


def kernel(x, c1_w, c1_scale, c1_shift, l1b0_c1_w, l1b0_c1_scale, l1b0_c1_shift, l1b0_c2_w, l1b0_c2_scale, l1b0_c2_shift, l1b0_c3_w, l1b0_c3_scale, l1b0_c3_shift, l1b0_ds_w, l1b0_ds_scale, l1b0_ds_shift, l1b1_c1_w, l1b1_c1_scale, l1b1_c1_shift, l1b1_c2_w, l1b1_c2_scale, l1b1_c2_shift, l1b1_c3_w, l1b1_c3_scale, l1b1_c3_shift, l1b2_c1_w, l1b2_c1_scale, l1b2_c1_shift, l1b2_c2_w, l1b2_c2_scale, l1b2_c2_shift, l1b2_c3_w, l1b2_c3_scale, l1b2_c3_shift, l2b0_c1_w, l2b0_c1_scale, l2b0_c1_shift, l2b0_c2_w, l2b0_c2_scale, l2b0_c2_shift, l2b0_c3_w, l2b0_c3_scale, l2b0_c3_shift, l2b0_ds_w, l2b0_ds_scale, l2b0_ds_shift, l2b1_c1_w, l2b1_c1_scale, l2b1_c1_shift, l2b1_c2_w, l2b1_c2_scale, l2b1_c2_shift, l2b1_c3_w, l2b1_c3_scale, l2b1_c3_shift, l2b2_c1_w, l2b2_c1_scale, l2b2_c1_shift, l2b2_c2_w, l2b2_c2_scale, l2b2_c2_shift, l2b2_c3_w, l2b2_c3_scale, l2b2_c3_shift, l2b3_c1_w, l2b3_c1_scale, l2b3_c1_shift, l2b3_c2_w, l2b3_c2_scale, l2b3_c2_shift, l2b3_c3_w, l2b3_c3_scale, l2b3_c3_shift, l3b0_c1_w, l3b0_c1_scale, l3b0_c1_shift, l3b0_c2_w, l3b0_c2_scale, l3b0_c2_shift, l3b0_c3_w, l3b0_c3_scale, l3b0_c3_shift, l3b0_ds_w, l3b0_ds_scale, l3b0_ds_shift, l3b1_c1_w, l3b1_c1_scale, l3b1_c1_shift, l3b1_c2_w, l3b1_c2_scale, l3b1_c2_shift, l3b1_c3_w, l3b1_c3_scale, l3b1_c3_shift, l3b2_c1_w, l3b2_c1_scale, l3b2_c1_shift, l3b2_c2_w, l3b2_c2_scale, l3b2_c2_shift, l3b2_c3_w, l3b2_c3_scale, l3b2_c3_shift, l3b3_c1_w, l3b3_c1_scale, l3b3_c1_shift, l3b3_c2_w, l3b3_c2_scale, l3b3_c2_shift, l3b3_c3_w, l3b3_c3_scale, l3b3_c3_shift, l3b4_c1_w, l3b4_c1_scale, l3b4_c1_shift, l3b4_c2_w, l3b4_c2_scale, l3b4_c2_shift, l3b4_c3_w, l3b4_c3_scale, l3b4_c3_shift, l3b5_c1_w, l3b5_c1_scale, l3b5_c1_shift, l3b5_c2_w, l3b5_c2_scale, l3b5_c2_shift, l3b5_c3_w, l3b5_c3_scale, l3b5_c3_shift, l4b0_c1_w, l4b0_c1_scale, l4b0_c1_shift, l4b0_c2_w, l4b0_c2_scale, l4b0_c2_shift, l4b0_c3_w, l4b0_c3_scale, l4b0_c3_shift, l4b0_ds_w, l4b0_ds_scale, l4b0_ds_shift, l4b1_c1_w, l4b1_c1_scale, l4b1_c1_shift, l4b1_c2_w, l4b1_c2_scale, l4b1_c2_shift, l4b1_c3_w, l4b1_c3_scale, l4b1_c3_shift, l4b2_c1_w, l4b2_c1_scale, l4b2_c1_shift, l4b2_c2_w, l4b2_c2_scale, l4b2_c2_shift, l4b2_c3_w, l4b2_c3_scale, l4b2_c3_shift, fc_w, fc_b):
    lv = locals()
    layer_blocks = [("l1", 3), ("l2", 4), ("l3", 6), ("l4", 3)]
    layers = []
    for lname, nblk in layer_blocks:
        blocks = []
        for b in range(nblk):
            pre = "%sb%d" % (lname, b)
            blk = {}
            for cn in ("c1", "c2", "c3"):
                blk[cn.replace("c", "conv")] = (
                    lv["%s_%s_w" % (pre, cn)],
                    lv["%s_%s_scale" % (pre, cn)],
                    lv["%s_%s_shift" % (pre, cn)])
            if ("%s_ds_w" % pre) in lv:
                blk["ds"] = (lv["%s_ds_w" % pre],
                             lv["%s_ds_scale" % pre],
                             lv["%s_ds_shift" % pre])
            blocks.append(blk)
        layers.append(blocks)

    h = jnp.transpose(x, (0, 2, 3, 1)).astype(jnp.bfloat16)
    h = _stem_conv(h, c1_w, c1_scale, c1_shift)
    h = _maxpool_3x3_s2(h)
    strides = [1, 2, 2, 2]
    for li, blocks in enumerate(layers):
        for b, blk in enumerate(blocks):
            h = _bottleneck(h, blk, strides[li] if b == 0 else 1)
    return _head(h, fc_w, fc_b)


# speedup vs baseline: 3.3527x; 1.0059x over previous
import functools
import jax
import jax.numpy as jnp
from jax.experimental import pallas as pl
from jax.experimental.pallas import tpu as pltpu

_VMEM_LIMIT = 32 * 1024 * 1024


def _ceil_to(x, m):
    return ((x + m - 1) // m) * m


def _mm_body(a_ref, w_ref, s_ref, t_ref, o_ref, *, act):
    y = jnp.dot(a_ref[...], w_ref[...], preferred_element_type=jnp.float32)
    y = y * s_ref[...] + t_ref[...]
    if act == "relu":
        y = jnp.maximum(y, 0.0)
    o_ref[...] = y.astype(o_ref.dtype)


def _mm_res_body(a_ref, w_ref, s_ref, t_ref, r_ref, o_ref, *, act):
    y = jnp.dot(a_ref[...], w_ref[...], preferred_element_type=jnp.float32)
    y = y * s_ref[...] + t_ref[...]
    y = y + r_ref[...].astype(jnp.float32)
    if act == "relu":
        y = jnp.maximum(y, 0.0)
    o_ref[...] = y.astype(o_ref.dtype)


def _mm(a, w, scale, shift, act="none", residual=None, out_dtype=jnp.bfloat16):
    M, K = a.shape
    N = w.shape[1]
    if M % 784 == 0:
        tm = 784
    else:
        tm = M
    tn = min(512, N)
    s2 = scale.astype(jnp.float32).reshape(1, N)
    t2 = shift.astype(jnp.float32).reshape(1, N)

    inputs = [a, w, s2, t2]
    in_specs = [
        pl.BlockSpec((tm, K), lambda i, j: (i, 0)),
        pl.BlockSpec((K, tn), lambda i, j: (0, j)),
        pl.BlockSpec((1, tn), lambda i, j: (0, j)),
        pl.BlockSpec((1, tn), lambda i, j: (0, j)),
    ]
    if residual is not None:
        body = functools.partial(_mm_res_body, act=act)
        inputs.append(residual)
        in_specs.append(pl.BlockSpec((tm, tn), lambda i, j: (i, j)))
    else:
        body = functools.partial(_mm_body, act=act)

    return pl.pallas_call(
        body,
        grid=(M // tm, N // tn),
        in_specs=in_specs,
        out_specs=pl.BlockSpec((tm, tn), lambda i, j: (i, j)),
        out_shape=jax.ShapeDtypeStruct((M, N), out_dtype),
        compiler_params=pltpu.CompilerParams(
            dimension_semantics=("parallel", "parallel"),
            vmem_limit_bytes=_VMEM_LIMIT),
    )(*inputs)


def _c3_body(x_ref, w_ref, s_ref, t_ref, o_ref, *, H, W, Wp, F, Fp):
    C = x_ref.shape[3]
    M2 = H * Wp
    p = jnp.dot(x_ref[0, 0:H, :, :].reshape(M2, C), w_ref[0],
                preferred_element_type=jnp.float32)
    p = p + jnp.dot(x_ref[0, 1:H + 1, :, :].reshape(M2, C), w_ref[1],
                    preferred_element_type=jnp.float32)
    p = p + jnp.dot(x_ref[0, 2:H + 2, :, :].reshape(M2, C), w_ref[2],
                    preferred_element_type=jnp.float32)
    p = p.reshape(H, Wp, 3 * Fp)
    acc = (p[:, 0:W, 0:Fp] + p[:, 1:W + 1, Fp:2 * Fp]
           + p[:, 2:W + 2, 2 * Fp:3 * Fp])
    y = jnp.maximum(acc * s_ref[...] + t_ref[...], 0.0)
    o_ref[0] = y[:, :, 0:F].astype(o_ref.dtype)


def _conv3_s1(x, w, scale, shift):
    Nb, H, W, C = x.shape
    F = w.shape[1]
    Wp = _ceil_to(W + 2, 16)
    Fp = max(F, 128)
    xp = jnp.pad(x, ((0, 0), (1, 1), (1, Wp - W - 1), (0, 0)))
    wt = jnp.transpose(w.reshape(3, 3, C, F), (0, 2, 1, 3))
    if Fp != F:
        wt = jnp.pad(wt, ((0, 0), (0, 0), (0, 0), (0, Fp - F)))
    ws = wt.reshape(3, C, 3 * Fp)
    sp = jnp.pad(scale.astype(jnp.float32), (0, Fp - F)).reshape(1, 1, Fp)
    tp = jnp.pad(shift.astype(jnp.float32), (0, Fp - F)).reshape(1, 1, Fp)

    return pl.pallas_call(
        functools.partial(_c3_body, H=H, W=W, Wp=Wp, F=F, Fp=Fp),
        grid=(Nb,),
        in_specs=[
            pl.BlockSpec((1, H + 2, Wp, C), lambda n: (n, 0, 0, 0)),
            pl.BlockSpec((3, C, 3 * Fp), lambda n: (0, 0, 0)),
            pl.BlockSpec((1, 1, Fp), lambda n: (0, 0, 0)),
            pl.BlockSpec((1, 1, Fp), lambda n: (0, 0, 0)),
        ],
        out_specs=pl.BlockSpec((1, H, W, F), lambda n: (n, 0, 0, 0)),
        out_shape=jax.ShapeDtypeStruct((Nb, H, W, F), jnp.bfloat16),
        compiler_params=pltpu.CompilerParams(
            dimension_semantics=("parallel",),
            vmem_limit_bytes=_VMEM_LIMIT),
    )(xp, ws, sp, tp)


def _stem_body(x_ref, w_ref, s_ref, t_ref, o_ref, *, Wo, Wp):
    th = x_ref.shape[1]
    A = x_ref[0].reshape(th * Wp, 48)
    p = jnp.dot(A, w_ref[...], preferred_element_type=jnp.float32)
    p = p.reshape(th, Wp, 256)
    acc = (p[:, 0:Wo, 0:64] + p[:, 1:Wo + 1, 64:128]
           + p[:, 2:Wo + 2, 128:192] + p[:, 3:Wo + 3, 192:256])
    y = jnp.maximum(acc * s_ref[...] + t_ref[...], 0.0)
    o_ref[0] = y.astype(o_ref.dtype)


def _stem_conv(xh, w, scale, shift):
    Nb, H, W, _ = xh.shape
    Ho, Wo = H // 2, W // 2
    Wp = _ceil_to(Wo + 3, 16)
    xp = jnp.pad(xh, ((0, 0), (3, 3), (3, 3), (0, 0)))
    planes = [jnp.pad(xp[:, a::2, b::2, :][:, :Ho + 3, :Wo + 3, :],
                      ((0, 0), (0, 0), (0, Wp - Wo - 3), (0, 0)))
              for a in (0, 1) for b in (0, 1)]
    xs = jnp.concatenate(planes, axis=-1)
    xss = jnp.concatenate([xs[:, j:j + Ho] for j in range(4)],
                          axis=-1)
    w6 = jnp.zeros((8, 8, 3, 64), jnp.bfloat16).at[:7, :7].set(
        w.reshape(7, 7, 3, 64))
    wc = jnp.transpose(w6.reshape(4, 2, 4, 2, 3, 64),
                       (0, 1, 3, 4, 2, 5)).reshape(48, 256)
    sp = scale.astype(jnp.float32).reshape(1, 1, 64)
    tp = shift.astype(jnp.float32).reshape(1, 1, 64)
    th = 28 if Ho % 28 == 0 else Ho
    return pl.pallas_call(
        functools.partial(_stem_body, Wo=Wo, Wp=Wp),
        grid=(Nb, Ho // th),
        in_specs=[
            pl.BlockSpec((1, th, Wp, 48), lambda n, i: (n, i, 0, 0)),
            pl.BlockSpec((48, 256), lambda n, i: (0, 0)),
            pl.BlockSpec((1, 1, 64), lambda n, i: (0, 0, 0)),
            pl.BlockSpec((1, 1, 64), lambda n, i: (0, 0, 0)),
        ],
        out_specs=pl.BlockSpec((1, th, Wo, 64), lambda n, i: (n, i, 0, 0)),
        out_shape=jax.ShapeDtypeStruct((Nb, Ho, Wo, 64), jnp.bfloat16),
        compiler_params=pltpu.CompilerParams(
            dimension_semantics=("parallel", "parallel"),
            vmem_limit_bytes=_VMEM_LIMIT),
    )(xss, wc, sp, tp)


def _mp_body(x_ref, o_ref):
    H = x_ref.shape[1]
    Wh = x_ref.shape[2]
    C = o_ref.shape[3]
    ninf = jnp.full((), float("-inf"), x_ref.dtype)
    xv = x_ref[0]
    a = xv[:, :, 0:C]
    b = xv[:, :, C:2 * C]
    bm = jnp.concatenate(
        [jnp.full((H, 1, C), ninf, xv.dtype), b[:, :Wh - 1, :]], axis=1)
    mw = jnp.maximum(jnp.maximum(a, b), bm)
    rm = jnp.concatenate([jnp.full((1, Wh, C), ninf, xv.dtype),
                          mw[:H - 1]], axis=0)
    rp = jnp.concatenate([mw[1:], jnp.full((1, Wh, C), ninf, xv.dtype)],
                         axis=0)
    m3 = jnp.maximum(jnp.maximum(mw, rm), rp)
    o_ref[0] = m3.reshape(H // 2, 2, Wh, C)[:, 0]


def _maxpool_3x3_s2(x):
    Nb, H, W, C = x.shape
    xv = x.reshape(Nb, H, W // 2, 2 * C)
    return pl.pallas_call(
        _mp_body,
        grid=(Nb,),
        in_specs=[pl.BlockSpec((1, H, W // 2, 2 * C), lambda n: (n, 0, 0, 0))],
        out_specs=pl.BlockSpec((1, H // 2, W // 2, C), lambda n: (n, 0, 0, 0)),
        out_shape=jax.ShapeDtypeStruct((Nb, H // 2, W // 2, C), x.dtype),
        compiler_params=pltpu.CompilerParams(
            dimension_semantics=("parallel",),
            vmem_limit_bytes=_VMEM_LIMIT),
    )(xv)


def _head_body(x_ref, w_ref, b_ref, o_ref, *, HW):
    xs = jnp.sum(x_ref[...].astype(jnp.float32), axis=1)
    pooled = (xs * (1.0 / HW)).astype(jnp.bfloat16).astype(jnp.float32)
    wv = w_ref[...].astype(jnp.float32)
    logit = jnp.sum(pooled * wv, axis=1, keepdims=True) + b_ref[...]
    o_ref[...] = 1.0 / (1.0 + jnp.exp(-logit))


def _head(x, fc_w, fc_b):
    Nb, H, W, C = x.shape
    x3 = x.reshape(Nb, H * W, C)
    wv = fc_w.reshape(1, C)
    bv = fc_b.astype(jnp.float32).reshape(1, 1)
    return pl.pallas_call(
        functools.partial(_head_body, HW=H * W),
        grid=(1,),
        in_specs=[
            pl.BlockSpec((Nb, H * W, C), lambda i: (0, 0, 0)),
            pl.BlockSpec((1, C), lambda i: (0, 0)),
            pl.BlockSpec((1, 1), lambda i: (0, 0)),
        ],
        out_specs=pl.BlockSpec((Nb, 1), lambda i: (0, 0)),
        out_shape=jax.ShapeDtypeStruct((Nb, 1), jnp.float32),
        compiler_params=pltpu.CompilerParams(
            dimension_semantics=("arbitrary",),
            vmem_limit_bytes=_VMEM_LIMIT),
    )(x3, wv, bv)


def _tail_body(*refs, H, W, Wp, P, C4, nblk):
    x_ref = refs[0]
    o_ref = refs[-1]
    Hp = H + 2
    Pp = max(P, 128)
    f32 = jnp.float32
    xc = x_ref[0]
    ri = jax.lax.broadcasted_iota(jnp.int32, (Hp, Wp, 1), 0)
    ci = jax.lax.broadcasted_iota(jnp.int32, (Hp, Wp, 1), 1)
    m2 = ((ri >= 1) & (ri <= H) & (ci >= 1) & (ci <= W)).reshape(Hp * Wp, 1)
    M2 = H * Wp
    for b in range(nblk):
        w1, s1, t1, w2, s2, t2, w3, s3, t3 = refs[1 + 9 * b:10 + 9 * b]
        x2 = xc.reshape(Hp * Wp, C4)
        u = jnp.dot(x2, w1[...], preferred_element_type=f32)
        u = u * s1[...] + t1[...]
        u = jnp.where(m2, jnp.maximum(u, 0.0), 0.0).astype(jnp.bfloat16)
        up = u.reshape(Hp, Wp, P)
        p = jnp.dot(up[0:H].reshape(M2, P), w2[0], preferred_element_type=f32)
        p = p + jnp.dot(up[1:H + 1].reshape(M2, P), w2[1],
                        preferred_element_type=f32)
        p = p + jnp.dot(up[2:H + 2].reshape(M2, P), w2[2],
                        preferred_element_type=f32)
        p = p.reshape(H, Wp, 3 * Pp)
        acc = (p[:, 0:W, 0:Pp] + p[:, 1:W + 1, Pp:2 * Pp]
               + p[:, 2:W + 2, 2 * Pp:3 * Pp])
        v = jnp.maximum(acc * s2[...] + t2[...], 0.0)[:, :, 0:P]
        vp = jnp.pad(v.astype(jnp.bfloat16), ((1, 1), (1, Wp - W - 1), (0, 0)))
        y = jnp.dot(vp.reshape(Hp * Wp, P), w3[...], preferred_element_type=f32)
        y = y * s3[...] + t3[...] + x2.astype(f32)
        y = jnp.where(m2, jnp.maximum(y, 0.0), 0.0).astype(jnp.bfloat16)
        xc = y.reshape(Hp, Wp, C4)
    o_ref[0] = xc[1:H + 1, 1:W + 1, :]


def _layer_tail(x, blks):
    Nb, H, W, C4 = x.shape
    P = blks[0]["conv1"][0].shape[1]
    Pp = max(P, 128)
    Wp = _ceil_to(W + 2, 16)
    xp = jnp.pad(x, ((0, 0), (1, 1), (1, Wp - W - 1), (0, 0)))
    inputs = [xp]
    in_specs = [pl.BlockSpec((1, H + 2, Wp, C4), lambda n: (n, 0, 0, 0))]

    def const_spec(shape):
        nd = len(shape)
        return pl.BlockSpec(shape, lambda n, _nd=nd: (0,) * _nd)

    for blk in blks:
        w1, s1, t1 = blk["conv1"]
        w2, s2, t2 = blk["conv2"]
        w3, s3, t3 = blk["conv3"]
        wt = jnp.transpose(w2.reshape(3, 3, P, P), (0, 2, 1, 3))
        if Pp != P:
            wt = jnp.pad(wt, ((0, 0), (0, 0), (0, 0), (0, Pp - P)))
        ws = wt.reshape(3, P, 3 * Pp)
        for arr in (w1, s1.astype(jnp.float32).reshape(1, P),
                    t1.astype(jnp.float32).reshape(1, P), ws,
                    jnp.pad(s2.astype(jnp.float32), (0, Pp - P)).reshape(1, 1, Pp),
                    jnp.pad(t2.astype(jnp.float32), (0, Pp - P)).reshape(1, 1, Pp),
                    w3, s3.astype(jnp.float32).reshape(1, C4),
                    t3.astype(jnp.float32).reshape(1, C4)):
            inputs.append(arr)
            in_specs.append(const_spec(arr.shape))

    return pl.pallas_call(
        functools.partial(_tail_body, H=H, W=W, Wp=Wp, P=P, C4=C4,
                          nblk=len(blks)),
        grid=(Nb,),
        in_specs=in_specs,
        out_specs=pl.BlockSpec((1, H, W, C4), lambda n: (n, 0, 0, 0)),
        out_shape=jax.ShapeDtypeStruct((Nb, H, W, C4), jnp.bfloat16),
        compiler_params=pltpu.CompilerParams(
            dimension_semantics=("parallel",),
            vmem_limit_bytes=_VMEM_LIMIT),
    )(*inputs)


def _im2col(x, k, stride, pad):
    Nb, H, W, C = x.shape
    Ho = (H + 2 * pad - k) // stride + 1
    Wo = (W + 2 * pad - k) // stride + 1
    xp = jnp.pad(x, ((0, 0), (pad, pad), (pad, pad), (0, 0)))
    cols = [xp[:, dy:dy + stride * (Ho - 1) + 1:stride,
               dx:dx + stride * (Wo - 1) + 1:stride, :]
            for dy in range(k) for dx in range(k)]
    patches = jnp.stack(cols, axis=3)
    return patches.reshape(Nb * Ho * Wo, k * k * C), (Nb, Ho, Wo)


def _bottleneck(x, blk, stride):
    Nb, H, W, Cin = x.shape
    x2d = x.reshape(-1, Cin)
    w1, s1, t1 = blk["conv1"]
    w2, s2, t2 = blk["conv2"]
    w3, s3, t3 = blk["conv3"]
    planes = w1.shape[1]

    u = _mm(x2d, w1, s1, t1, act="relu")
    u = u.reshape(Nb, H, W, planes)
    if stride == 1:
        v = _conv3_s1(u, w2, s2, t2)
    else:
        a, (nb, ho, wo) = _im2col(u, 3, stride, 1)
        v = _mm(a, w2, s2, t2, act="relu").reshape(nb, ho, wo, planes)
    Ho, Wo = v.shape[1], v.shape[2]

    if "ds" in blk:
        wd, sd, td = blk["ds"]
        xs = x[:, ::stride, ::stride, :] if stride > 1 else x
        ident = _mm(xs.reshape(-1, Cin), wd, sd, td, act="none")
    else:
        ident = x2d
    out = _mm(v.reshape(-1, planes), w3, s3, t3, act="relu", residual=ident)
    return out.reshape(Nb, Ho, Wo, 4 * planes)


def kernel(x, c1_w, c1_scale, c1_shift, l1b0_c1_w, l1b0_c1_scale, l1b0_c1_shift, l1b0_c2_w, l1b0_c2_scale, l1b0_c2_shift, l1b0_c3_w, l1b0_c3_scale, l1b0_c3_shift, l1b0_ds_w, l1b0_ds_scale, l1b0_ds_shift, l1b1_c1_w, l1b1_c1_scale, l1b1_c1_shift, l1b1_c2_w, l1b1_c2_scale, l1b1_c2_shift, l1b1_c3_w, l1b1_c3_scale, l1b1_c3_shift, l1b2_c1_w, l1b2_c1_scale, l1b2_c1_shift, l1b2_c2_w, l1b2_c2_scale, l1b2_c2_shift, l1b2_c3_w, l1b2_c3_scale, l1b2_c3_shift, l2b0_c1_w, l2b0_c1_scale, l2b0_c1_shift, l2b0_c2_w, l2b0_c2_scale, l2b0_c2_shift, l2b0_c3_w, l2b0_c3_scale, l2b0_c3_shift, l2b0_ds_w, l2b0_ds_scale, l2b0_ds_shift, l2b1_c1_w, l2b1_c1_scale, l2b1_c1_shift, l2b1_c2_w, l2b1_c2_scale, l2b1_c2_shift, l2b1_c3_w, l2b1_c3_scale, l2b1_c3_shift, l2b2_c1_w, l2b2_c1_scale, l2b2_c1_shift, l2b2_c2_w, l2b2_c2_scale, l2b2_c2_shift, l2b2_c3_w, l2b2_c3_scale, l2b2_c3_shift, l2b3_c1_w, l2b3_c1_scale, l2b3_c1_shift, l2b3_c2_w, l2b3_c2_scale, l2b3_c2_shift, l2b3_c3_w, l2b3_c3_scale, l2b3_c3_shift, l3b0_c1_w, l3b0_c1_scale, l3b0_c1_shift, l3b0_c2_w, l3b0_c2_scale, l3b0_c2_shift, l3b0_c3_w, l3b0_c3_scale, l3b0_c3_shift, l3b0_ds_w, l3b0_ds_scale, l3b0_ds_shift, l3b1_c1_w, l3b1_c1_scale, l3b1_c1_shift, l3b1_c2_w, l3b1_c2_scale, l3b1_c2_shift, l3b1_c3_w, l3b1_c3_scale, l3b1_c3_shift, l3b2_c1_w, l3b2_c1_scale, l3b2_c1_shift, l3b2_c2_w, l3b2_c2_scale, l3b2_c2_shift, l3b2_c3_w, l3b2_c3_scale, l3b2_c3_shift, l3b3_c1_w, l3b3_c1_scale, l3b3_c1_shift, l3b3_c2_w, l3b3_c2_scale, l3b3_c2_shift, l3b3_c3_w, l3b3_c3_scale, l3b3_c3_shift, l3b4_c1_w, l3b4_c1_scale, l3b4_c1_shift, l3b4_c2_w, l3b4_c2_scale, l3b4_c2_shift, l3b4_c3_w, l3b4_c3_scale, l3b4_c3_shift, l3b5_c1_w, l3b5_c1_scale, l3b5_c1_shift, l3b5_c2_w, l3b5_c2_scale, l3b5_c2_shift, l3b5_c3_w, l3b5_c3_scale, l3b5_c3_shift, l4b0_c1_w, l4b0_c1_scale, l4b0_c1_shift, l4b0_c2_w, l4b0_c2_scale, l4b0_c2_shift, l4b0_c3_w, l4b0_c3_scale, l4b0_c3_shift, l4b0_ds_w, l4b0_ds_scale, l4b0_ds_shift, l4b1_c1_w, l4b1_c1_scale, l4b1_c1_shift, l4b1_c2_w, l4b1_c2_scale, l4b1_c2_shift, l4b1_c3_w, l4b1_c3_scale, l4b1_c3_shift, l4b2_c1_w, l4b2_c1_scale, l4b2_c1_shift, l4b2_c2_w, l4b2_c2_scale, l4b2_c2_shift, l4b2_c3_w, l4b2_c3_scale, l4b2_c3_shift, fc_w, fc_b):
    lv = locals()
    layer_blocks = [("l1", 3), ("l2", 4), ("l3", 6), ("l4", 3)]
    layers = []
    for lname, nblk in layer_blocks:
        blocks = []
        for b in range(nblk):
            pre = "%sb%d" % (lname, b)
            blk = {}
            for cn in ("c1", "c2", "c3"):
                blk[cn.replace("c", "conv")] = (
                    lv["%s_%s_w" % (pre, cn)],
                    lv["%s_%s_scale" % (pre, cn)],
                    lv["%s_%s_shift" % (pre, cn)])
            if ("%s_ds_w" % pre) in lv:
                blk["ds"] = (lv["%s_ds_w" % pre],
                             lv["%s_ds_scale" % pre],
                             lv["%s_ds_shift" % pre])
            blocks.append(blk)
        layers.append(blocks)

    h = jnp.transpose(x, (0, 2, 3, 1)).astype(jnp.bfloat16)
    h = _stem_conv(h, c1_w, c1_scale, c1_shift)
    h = _maxpool_3x3_s2(h)
    strides = [1, 2, 2, 2]
    for li, blocks in enumerate(layers):
        h = _bottleneck(h, blocks[0], strides[li])
        h = _layer_tail(h, blocks[1:])
    return _head(h, fc_w, fc_b)


# speedup vs baseline: 7.4222x; 2.2138x over previous
import functools
import jax
import jax.numpy as jnp
from jax.experimental import pallas as pl
from jax.experimental.pallas import tpu as pltpu

_VMEM_LIMIT = 32 * 1024 * 1024


def _ceil_to(x, m):
    return ((x + m - 1) // m) * m


def _mm_body(a_ref, w_ref, s_ref, t_ref, o_ref, *, act):
    y = jnp.dot(a_ref[...], w_ref[...], preferred_element_type=jnp.float32)
    y = y * s_ref[...] + t_ref[...]
    if act == "relu":
        y = jnp.maximum(y, 0.0)
    o_ref[...] = y.astype(o_ref.dtype)


def _mm_res_body(a_ref, w_ref, s_ref, t_ref, r_ref, o_ref, *, act):
    y = jnp.dot(a_ref[...], w_ref[...], preferred_element_type=jnp.float32)
    y = y * s_ref[...] + t_ref[...]
    y = y + r_ref[...].astype(jnp.float32)
    if act == "relu":
        y = jnp.maximum(y, 0.0)
    o_ref[...] = y.astype(o_ref.dtype)


def _mm(a, w, scale, shift, act="none", residual=None, out_dtype=jnp.bfloat16):
    M, K = a.shape
    N = w.shape[1]
    if M % 784 == 0:
        tm = 784
    else:
        tm = M
    tn = min(512, N)
    s2 = scale.astype(jnp.float32).reshape(1, N)
    t2 = shift.astype(jnp.float32).reshape(1, N)

    inputs = [a, w, s2, t2]
    in_specs = [
        pl.BlockSpec((tm, K), lambda i, j: (i, 0)),
        pl.BlockSpec((K, tn), lambda i, j: (0, j)),
        pl.BlockSpec((1, tn), lambda i, j: (0, j)),
        pl.BlockSpec((1, tn), lambda i, j: (0, j)),
    ]
    if residual is not None:
        body = functools.partial(_mm_res_body, act=act)
        inputs.append(residual)
        in_specs.append(pl.BlockSpec((tm, tn), lambda i, j: (i, j)))
    else:
        body = functools.partial(_mm_body, act=act)

    return pl.pallas_call(
        body,
        grid=(M // tm, N // tn),
        in_specs=in_specs,
        out_specs=pl.BlockSpec((tm, tn), lambda i, j: (i, j)),
        out_shape=jax.ShapeDtypeStruct((M, N), out_dtype),
        compiler_params=pltpu.CompilerParams(
            dimension_semantics=("parallel", "parallel"),
            vmem_limit_bytes=_VMEM_LIMIT),
    )(*inputs)


def _c3_body(x_ref, w_ref, s_ref, t_ref, o_ref, *, H, W, Wp, F, Fp):
    C = x_ref.shape[3]
    M2 = H * Wp
    p = jnp.dot(x_ref[0, 0:H, :, :].reshape(M2, C), w_ref[0],
                preferred_element_type=jnp.float32)
    p = p + jnp.dot(x_ref[0, 1:H + 1, :, :].reshape(M2, C), w_ref[1],
                    preferred_element_type=jnp.float32)
    p = p + jnp.dot(x_ref[0, 2:H + 2, :, :].reshape(M2, C), w_ref[2],
                    preferred_element_type=jnp.float32)
    p = p.reshape(H, Wp, 3 * Fp)
    acc = (p[:, 0:W, 0:Fp] + p[:, 1:W + 1, Fp:2 * Fp]
           + p[:, 2:W + 2, 2 * Fp:3 * Fp])
    y = jnp.maximum(acc * s_ref[...] + t_ref[...], 0.0)
    o_ref[0] = y[:, :, 0:F].astype(o_ref.dtype)


def _conv3_s1(x, w, scale, shift):
    Nb, H, W, C = x.shape
    F = w.shape[1]
    Wp = _ceil_to(W + 2, 16)
    Fp = max(F, 128)
    xp = jnp.pad(x, ((0, 0), (1, 1), (1, Wp - W - 1), (0, 0)))
    wt = jnp.transpose(w.reshape(3, 3, C, F), (0, 2, 1, 3))
    if Fp != F:
        wt = jnp.pad(wt, ((0, 0), (0, 0), (0, 0), (0, Fp - F)))
    ws = wt.reshape(3, C, 3 * Fp)
    sp = jnp.pad(scale.astype(jnp.float32), (0, Fp - F)).reshape(1, 1, Fp)
    tp = jnp.pad(shift.astype(jnp.float32), (0, Fp - F)).reshape(1, 1, Fp)

    return pl.pallas_call(
        functools.partial(_c3_body, H=H, W=W, Wp=Wp, F=F, Fp=Fp),
        grid=(Nb,),
        in_specs=[
            pl.BlockSpec((1, H + 2, Wp, C), lambda n: (n, 0, 0, 0)),
            pl.BlockSpec((3, C, 3 * Fp), lambda n: (0, 0, 0)),
            pl.BlockSpec((1, 1, Fp), lambda n: (0, 0, 0)),
            pl.BlockSpec((1, 1, Fp), lambda n: (0, 0, 0)),
        ],
        out_specs=pl.BlockSpec((1, H, W, F), lambda n: (n, 0, 0, 0)),
        out_shape=jax.ShapeDtypeStruct((Nb, H, W, F), jnp.bfloat16),
        compiler_params=pltpu.CompilerParams(
            dimension_semantics=("parallel",),
            vmem_limit_bytes=_VMEM_LIMIT),
    )(xp, ws, sp, tp)


def _stem_body(x_ref, w_ref, s_ref, t_ref, o_ref, *, Wo, Wp):
    th = x_ref.shape[1]
    A = x_ref[0].reshape(th * Wp, 48)
    p = jnp.dot(A, w_ref[...], preferred_element_type=jnp.float32)
    p = p.reshape(th, Wp, 256)
    acc = (p[:, 0:Wo, 0:64] + p[:, 1:Wo + 1, 64:128]
           + p[:, 2:Wo + 2, 128:192] + p[:, 3:Wo + 3, 192:256])
    y = jnp.maximum(acc * s_ref[...] + t_ref[...], 0.0)
    o_ref[0] = y.astype(o_ref.dtype)


def _stem_conv(xh, w, scale, shift):
    Nb, H, W, _ = xh.shape
    Ho, Wo = H // 2, W // 2
    Wp = _ceil_to(Wo + 3, 16)
    xp = jnp.pad(xh, ((0, 0), (3, 3), (3, 3), (0, 0)))
    planes = [jnp.pad(xp[:, a::2, b::2, :][:, :Ho + 3, :Wo + 3, :],
                      ((0, 0), (0, 0), (0, Wp - Wo - 3), (0, 0)))
              for a in (0, 1) for b in (0, 1)]
    xs = jnp.concatenate(planes, axis=-1)
    xss = jnp.concatenate([xs[:, j:j + Ho] for j in range(4)],
                          axis=-1)
    w6 = jnp.zeros((8, 8, 3, 64), jnp.bfloat16).at[:7, :7].set(
        w.reshape(7, 7, 3, 64))
    wc = jnp.transpose(w6.reshape(4, 2, 4, 2, 3, 64),
                       (0, 1, 3, 4, 2, 5)).reshape(48, 256)
    sp = scale.astype(jnp.float32).reshape(1, 1, 64)
    tp = shift.astype(jnp.float32).reshape(1, 1, 64)
    th = 28 if Ho % 28 == 0 else Ho
    return pl.pallas_call(
        functools.partial(_stem_body, Wo=Wo, Wp=Wp),
        grid=(Nb, Ho // th),
        in_specs=[
            pl.BlockSpec((1, th, Wp, 48), lambda n, i: (n, i, 0, 0)),
            pl.BlockSpec((48, 256), lambda n, i: (0, 0)),
            pl.BlockSpec((1, 1, 64), lambda n, i: (0, 0, 0)),
            pl.BlockSpec((1, 1, 64), lambda n, i: (0, 0, 0)),
        ],
        out_specs=pl.BlockSpec((1, th, Wo, 64), lambda n, i: (n, i, 0, 0)),
        out_shape=jax.ShapeDtypeStruct((Nb, Ho, Wo, 64), jnp.bfloat16),
        compiler_params=pltpu.CompilerParams(
            dimension_semantics=("parallel", "parallel"),
            vmem_limit_bytes=_VMEM_LIMIT),
    )(xss, wc, sp, tp)


def _mp_body(x_ref, o_ref):
    H = x_ref.shape[1]
    Wh = x_ref.shape[2]
    C = o_ref.shape[3]
    ninf = jnp.full((), float("-inf"), x_ref.dtype)
    xv = x_ref[0]
    a = xv[:, :, 0:C]
    b = xv[:, :, C:2 * C]
    bm = jnp.concatenate(
        [jnp.full((H, 1, C), ninf, xv.dtype), b[:, :Wh - 1, :]], axis=1)
    mw = jnp.maximum(jnp.maximum(a, b), bm)
    rm = jnp.concatenate([jnp.full((1, Wh, C), ninf, xv.dtype),
                          mw[:H - 1]], axis=0)
    rp = jnp.concatenate([mw[1:], jnp.full((1, Wh, C), ninf, xv.dtype)],
                         axis=0)
    m3 = jnp.maximum(jnp.maximum(mw, rm), rp)
    o_ref[0] = m3.reshape(H // 2, 2, Wh, C)[:, 0]


def _maxpool_3x3_s2(x):
    Nb, H, W, C = x.shape
    xv = x.reshape(Nb, H, W // 2, 2 * C)
    return pl.pallas_call(
        _mp_body,
        grid=(Nb,),
        in_specs=[pl.BlockSpec((1, H, W // 2, 2 * C), lambda n: (n, 0, 0, 0))],
        out_specs=pl.BlockSpec((1, H // 2, W // 2, C), lambda n: (n, 0, 0, 0)),
        out_shape=jax.ShapeDtypeStruct((Nb, H // 2, W // 2, C), x.dtype),
        compiler_params=pltpu.CompilerParams(
            dimension_semantics=("parallel",),
            vmem_limit_bytes=_VMEM_LIMIT),
    )(xv)


def _head_body(x_ref, w_ref, b_ref, o_ref, *, HW):
    xs = jnp.sum(x_ref[...].astype(jnp.float32), axis=1)
    pooled = (xs * (1.0 / HW)).astype(jnp.bfloat16).astype(jnp.float32)
    wv = w_ref[...].astype(jnp.float32)
    logit = jnp.sum(pooled * wv, axis=1, keepdims=True) + b_ref[...]
    o_ref[...] = 1.0 / (1.0 + jnp.exp(-logit))


def _head(x, fc_w, fc_b):
    Nb, H, W, C = x.shape
    x3 = x.reshape(Nb, H * W, C)
    wv = fc_w.reshape(1, C)
    bv = fc_b.astype(jnp.float32).reshape(1, 1)
    return pl.pallas_call(
        functools.partial(_head_body, HW=H * W),
        grid=(1,),
        in_specs=[
            pl.BlockSpec((Nb, H * W, C), lambda i: (0, 0, 0)),
            pl.BlockSpec((1, C), lambda i: (0, 0)),
            pl.BlockSpec((1, 1), lambda i: (0, 0)),
        ],
        out_specs=pl.BlockSpec((Nb, 1), lambda i: (0, 0)),
        out_shape=jax.ShapeDtypeStruct((Nb, 1), jnp.float32),
        compiler_params=pltpu.CompilerParams(
            dimension_semantics=("arbitrary",),
            vmem_limit_bytes=_VMEM_LIMIT),
    )(x3, wv, bv)


def _tail_body(*refs, H, W, Wp, P, C4, nblk, paired):
    x_ref = refs[0]
    o_ref = refs[-1]
    Hp = H + 2
    Pp = max(P, 128)
    f32 = jnp.float32
    x0 = x_ref[0][:, :, 0:C4] if paired else x_ref[0]
    xc = jnp.pad(x0, ((1, 1), (1, Wp - W - 1), (0, 0)))
    ri = jax.lax.broadcasted_iota(jnp.int32, (Hp, Wp, 1), 0)
    ci = jax.lax.broadcasted_iota(jnp.int32, (Hp, Wp, 1), 1)
    m2 = ((ri >= 1) & (ri <= H) & (ci >= 1) & (ci <= W)).reshape(Hp * Wp, 1)
    M2 = H * Wp
    for b in range(nblk):
        w1, s1, t1, w2, s2, t2, w3, s3, t3 = refs[1 + 9 * b:10 + 9 * b]
        x2 = xc.reshape(Hp * Wp, C4)
        u = jnp.dot(x2, w1[...], preferred_element_type=f32)
        u = u * s1[...] + t1[...]
        u = jnp.where(m2, jnp.maximum(u, 0.0), 0.0).astype(jnp.bfloat16)
        up = u.reshape(Hp, Wp, P)
        p = jnp.dot(up[0:H].reshape(M2, P), w2[0], preferred_element_type=f32)
        p = p + jnp.dot(up[1:H + 1].reshape(M2, P), w2[1],
                        preferred_element_type=f32)
        p = p + jnp.dot(up[2:H + 2].reshape(M2, P), w2[2],
                        preferred_element_type=f32)
        p = p.reshape(H, Wp, 3 * Pp)
        acc = (p[:, 0:W, 0:Pp] + p[:, 1:W + 1, Pp:2 * Pp]
               + p[:, 2:W + 2, 2 * Pp:3 * Pp])
        v = jnp.maximum(acc * s2[...] + t2[...], 0.0)[:, :, 0:P]
        vp = jnp.pad(v.astype(jnp.bfloat16), ((1, 1), (1, Wp - W - 1), (0, 0)))
        y = jnp.dot(vp.reshape(Hp * Wp, P), w3[...], preferred_element_type=f32)
        y = y * s3[...] + t3[...] + x2.astype(f32)
        y = jnp.where(m2, jnp.maximum(y, 0.0), 0.0).astype(jnp.bfloat16)
        xc = y.reshape(Hp, Wp, C4)
    o_ref[0] = xc[1:H + 1, 1:W + 1, :]


def _layer_tail(x, blks, paired):
    Nb, H, W, Cb = x.shape
    C4 = Cb // 2 if paired else Cb
    P = blks[0]["conv1"][0].shape[1]
    Pp = max(P, 128)
    Wp = _ceil_to(W + 2, 16)
    inputs = [x]
    in_specs = [pl.BlockSpec((1, H, W, Cb), lambda n: (n, 0, 0, 0))]

    def const_spec(shape):
        nd = len(shape)
        return pl.BlockSpec(shape, lambda n, _nd=nd: (0,) * _nd)

    for blk in blks:
        w1, s1, t1 = blk["conv1"]
        w2, s2, t2 = blk["conv2"]
        w3, s3, t3 = blk["conv3"]
        wt = jnp.transpose(w2.reshape(3, 3, P, P), (0, 2, 1, 3))
        if Pp != P:
            wt = jnp.pad(wt, ((0, 0), (0, 0), (0, 0), (0, Pp - P)))
        ws = wt.reshape(3, P, 3 * Pp)
        for arr in (w1, s1.astype(jnp.float32).reshape(1, P),
                    t1.astype(jnp.float32).reshape(1, P), ws,
                    jnp.pad(s2.astype(jnp.float32), (0, Pp - P)).reshape(1, 1, Pp),
                    jnp.pad(t2.astype(jnp.float32), (0, Pp - P)).reshape(1, 1, Pp),
                    w3, s3.astype(jnp.float32).reshape(1, C4),
                    t3.astype(jnp.float32).reshape(1, C4)):
            inputs.append(arr)
            in_specs.append(const_spec(arr.shape))

    return pl.pallas_call(
        functools.partial(_tail_body, H=H, W=W, Wp=Wp, P=P, C4=C4,
                          nblk=len(blks), paired=paired),
        grid=(Nb,),
        in_specs=in_specs,
        out_specs=pl.BlockSpec((1, H, W, C4), lambda n: (n, 0, 0, 0)),
        out_shape=jax.ShapeDtypeStruct((Nb, H, W, C4), jnp.bfloat16),
        compiler_params=pltpu.CompilerParams(
            dimension_semantics=("parallel",),
            vmem_limit_bytes=_VMEM_LIMIT),
    )(*inputs)


def _b0_body(x_ref, w1, s1, t1, w2, s2, t2, w3, s3, t3, wd, sd, td, o_ref,
             *, H, W, Wp, P, Cin, C4, stride):
    f32 = jnp.float32
    Hp = H + 2
    Pp = max(P, 128)
    Ho = H // stride
    Wq = _ceil_to(W, 16)
    xv = x_ref[0]
    xp = jnp.pad(xv, ((1, 1), (1, Wp - W - 1), (0, 0)))
    ri = jax.lax.broadcasted_iota(jnp.int32, (Hp, Wp, 1), 0)
    ci = jax.lax.broadcasted_iota(jnp.int32, (Hp, Wp, 1), 1)
    m2 = ((ri >= 1) & (ri <= H) & (ci >= 1) & (ci <= W)).reshape(Hp * Wp, 1)
    u = jnp.dot(xp.reshape(Hp * Wp, Cin), w1[...], preferred_element_type=f32)
    u = u * s1[...] + t1[...]
    u = jnp.where(m2, jnp.maximum(u, 0.0), 0.0).astype(jnp.bfloat16)
    up = u.reshape(Hp, Wp, P)
    M2 = Ho * Wp
    if stride == 1:
        rows = [up[dy:dy + H] for dy in range(3)]
    else:
        pv = up.reshape(Hp // 2, 2, Wp, P)
        rows = [pv[:, 0][0:Ho], pv[:, 1][0:Ho], pv[:, 0][1:Ho + 1]]
    p = jnp.dot(rows[0].reshape(M2, P), w2[0], preferred_element_type=f32)
    p = p + jnp.dot(rows[1].reshape(M2, P), w2[1], preferred_element_type=f32)
    p = p + jnp.dot(rows[2].reshape(M2, P), w2[2], preferred_element_type=f32)
    p = p.reshape(Ho, Wp, 3 * Pp)
    acc = (p[:, 0:W, 0:Pp] + p[:, 1:W + 1, Pp:2 * Pp]
           + p[:, 2:W + 2, 2 * Pp:3 * Pp])
    v = jnp.maximum(acc * s2[...] + t2[...], 0.0)[:, :, 0:P]
    vq = jnp.pad(v.astype(jnp.bfloat16), ((0, 0), (0, Wq - W), (0, 0)))
    if stride == 1:
        xe = xv
    else:
        xe = xv.reshape(H // 2, 2, W, Cin)[:, 0]
    xq = jnp.pad(xe, ((0, 0), (0, Wq - W), (0, 0)))
    d = jnp.dot(xq.reshape(Ho * Wq, Cin), wd[...], preferred_element_type=f32)
    d = d * sd[...] + td[...]
    y = jnp.dot(vq.reshape(Ho * Wq, P), w3[...], preferred_element_type=f32)
    y = y * s3[...] + t3[...] + d
    y = jnp.maximum(y, 0.0).astype(jnp.bfloat16).reshape(Ho, Wq, C4)
    o_ref[0] = y[:, 0:W, :]


def _b0_block(x, blk, stride):
    Nb, H, W, Cin = x.shape
    w1, s1, t1 = blk["conv1"]
    w2, s2, t2 = blk["conv2"]
    w3, s3, t3 = blk["conv3"]
    wd, sd, td = blk["ds"]
    P = w1.shape[1]
    C4 = w3.shape[1]
    Pp = max(P, 128)
    Wp = _ceil_to(W + 2, 16)
    wt = jnp.transpose(w2.reshape(3, 3, P, P), (0, 2, 1, 3))
    if Pp != P:
        wt = jnp.pad(wt, ((0, 0), (0, 0), (0, 0), (0, Pp - P)))
    ws = wt.reshape(3, P, 3 * Pp)
    inputs = [x, w1,
              s1.astype(jnp.float32).reshape(1, P),
              t1.astype(jnp.float32).reshape(1, P), ws,
              jnp.pad(s2.astype(jnp.float32), (0, Pp - P)).reshape(1, 1, Pp),
              jnp.pad(t2.astype(jnp.float32), (0, Pp - P)).reshape(1, 1, Pp),
              w3, s3.astype(jnp.float32).reshape(1, C4),
              t3.astype(jnp.float32).reshape(1, C4),
              wd, sd.astype(jnp.float32).reshape(1, C4),
              td.astype(jnp.float32).reshape(1, C4)]
    in_specs = [pl.BlockSpec((1, H, W, Cin), lambda n: (n, 0, 0, 0))]
    for arr in inputs[1:]:
        nd = len(arr.shape)
        in_specs.append(pl.BlockSpec(arr.shape, lambda n, _nd=nd: (0,) * _nd))
    Ho = H // stride
    return pl.pallas_call(
        functools.partial(_b0_body, H=H, W=W, Wp=Wp, P=P, Cin=Cin, C4=C4,
                          stride=stride),
        grid=(Nb,),
        in_specs=in_specs,
        out_specs=pl.BlockSpec((1, Ho, W, C4), lambda n: (n, 0, 0, 0)),
        out_shape=jax.ShapeDtypeStruct((Nb, Ho, W, C4), jnp.bfloat16),
        compiler_params=pltpu.CompilerParams(
            dimension_semantics=("parallel",),
            vmem_limit_bytes=_VMEM_LIMIT),
    )(*inputs)


def kernel(x, c1_w, c1_scale, c1_shift, l1b0_c1_w, l1b0_c1_scale, l1b0_c1_shift, l1b0_c2_w, l1b0_c2_scale, l1b0_c2_shift, l1b0_c3_w, l1b0_c3_scale, l1b0_c3_shift, l1b0_ds_w, l1b0_ds_scale, l1b0_ds_shift, l1b1_c1_w, l1b1_c1_scale, l1b1_c1_shift, l1b1_c2_w, l1b1_c2_scale, l1b1_c2_shift, l1b1_c3_w, l1b1_c3_scale, l1b1_c3_shift, l1b2_c1_w, l1b2_c1_scale, l1b2_c1_shift, l1b2_c2_w, l1b2_c2_scale, l1b2_c2_shift, l1b2_c3_w, l1b2_c3_scale, l1b2_c3_shift, l2b0_c1_w, l2b0_c1_scale, l2b0_c1_shift, l2b0_c2_w, l2b0_c2_scale, l2b0_c2_shift, l2b0_c3_w, l2b0_c3_scale, l2b0_c3_shift, l2b0_ds_w, l2b0_ds_scale, l2b0_ds_shift, l2b1_c1_w, l2b1_c1_scale, l2b1_c1_shift, l2b1_c2_w, l2b1_c2_scale, l2b1_c2_shift, l2b1_c3_w, l2b1_c3_scale, l2b1_c3_shift, l2b2_c1_w, l2b2_c1_scale, l2b2_c1_shift, l2b2_c2_w, l2b2_c2_scale, l2b2_c2_shift, l2b2_c3_w, l2b2_c3_scale, l2b2_c3_shift, l2b3_c1_w, l2b3_c1_scale, l2b3_c1_shift, l2b3_c2_w, l2b3_c2_scale, l2b3_c2_shift, l2b3_c3_w, l2b3_c3_scale, l2b3_c3_shift, l3b0_c1_w, l3b0_c1_scale, l3b0_c1_shift, l3b0_c2_w, l3b0_c2_scale, l3b0_c2_shift, l3b0_c3_w, l3b0_c3_scale, l3b0_c3_shift, l3b0_ds_w, l3b0_ds_scale, l3b0_ds_shift, l3b1_c1_w, l3b1_c1_scale, l3b1_c1_shift, l3b1_c2_w, l3b1_c2_scale, l3b1_c2_shift, l3b1_c3_w, l3b1_c3_scale, l3b1_c3_shift, l3b2_c1_w, l3b2_c1_scale, l3b2_c1_shift, l3b2_c2_w, l3b2_c2_scale, l3b2_c2_shift, l3b2_c3_w, l3b2_c3_scale, l3b2_c3_shift, l3b3_c1_w, l3b3_c1_scale, l3b3_c1_shift, l3b3_c2_w, l3b3_c2_scale, l3b3_c2_shift, l3b3_c3_w, l3b3_c3_scale, l3b3_c3_shift, l3b4_c1_w, l3b4_c1_scale, l3b4_c1_shift, l3b4_c2_w, l3b4_c2_scale, l3b4_c2_shift, l3b4_c3_w, l3b4_c3_scale, l3b4_c3_shift, l3b5_c1_w, l3b5_c1_scale, l3b5_c1_shift, l3b5_c2_w, l3b5_c2_scale, l3b5_c2_shift, l3b5_c3_w, l3b5_c3_scale, l3b5_c3_shift, l4b0_c1_w, l4b0_c1_scale, l4b0_c1_shift, l4b0_c2_w, l4b0_c2_scale, l4b0_c2_shift, l4b0_c3_w, l4b0_c3_scale, l4b0_c3_shift, l4b0_ds_w, l4b0_ds_scale, l4b0_ds_shift, l4b1_c1_w, l4b1_c1_scale, l4b1_c1_shift, l4b1_c2_w, l4b1_c2_scale, l4b1_c2_shift, l4b1_c3_w, l4b1_c3_scale, l4b1_c3_shift, l4b2_c1_w, l4b2_c1_scale, l4b2_c1_shift, l4b2_c2_w, l4b2_c2_scale, l4b2_c2_shift, l4b2_c3_w, l4b2_c3_scale, l4b2_c3_shift, fc_w, fc_b):
    lv = locals()
    layer_blocks = [("l1", 3), ("l2", 4), ("l3", 6), ("l4", 3)]
    layers = []
    for lname, nblk in layer_blocks:
        blocks = []
        for b in range(nblk):
            pre = "%sb%d" % (lname, b)
            blk = {}
            for cn in ("c1", "c2", "c3"):
                blk[cn.replace("c", "conv")] = (
                    lv["%s_%s_w" % (pre, cn)],
                    lv["%s_%s_scale" % (pre, cn)],
                    lv["%s_%s_shift" % (pre, cn)])
            if ("%s_ds_w" % pre) in lv:
                blk["ds"] = (lv["%s_ds_w" % pre],
                             lv["%s_ds_scale" % pre],
                             lv["%s_ds_shift" % pre])
            blocks.append(blk)
        layers.append(blocks)

    h = jnp.transpose(x, (0, 2, 3, 1)).astype(jnp.bfloat16)
    h = _stem_conv(h, c1_w, c1_scale, c1_shift)
    h = _maxpool_3x3_s2(h)
    strides = [1, 2, 2, 2]
    for li, blocks in enumerate(layers):
        s = strides[li]
        h = _b0_block(h, blocks[0], s)
        if s == 2:
            Nb, Ho, Wd, C4 = h.shape
            h = h.reshape(Nb, Ho, Wd // 2, 2 * C4)
        h = _layer_tail(h, blocks[1:], paired=(s == 2))
    return _head(h, fc_w, fc_b)
